# Initial kernel scaffold; baseline (speedup 1.0000x reference)
import jax
import jax.numpy as jnp
from jax.experimental import pallas as pl


def kernel(h, x, edge_index, edge_dist, W_e1, b_e1, W_e2, b_e2, W_a, b_a, W_n1, b_n1, W_n2, b_n2, W_c1, b_c1, W_c2, ln_g, ln_b):
    raise NotImplementedError("write your pallas kernel here")


# speedup vs baseline: 2.5463x; 2.5463x over previous
import functools

import jax
import jax.numpy as jnp
from jax import lax
from jax.experimental import pallas as pl
from jax.experimental.pallas import tpu as pltpu
from jax.experimental.pallas import tpu_sc as plsc

F32 = jnp.float32
I32 = jnp.int32

NC = 2
NS = 16
NW = NC * NS
CHUNK = 128


def _silu(v):
    return v * jax.nn.sigmoid(v)


def _pre_body(hp_ref, wr_ref, wc_ref, pr_ref, pc_ref):
    hblk = hp_ref[...]
    pr_ref[...] = jnp.dot(hblk, wr_ref[...], preferred_element_type=F32)
    pc_ref[...] = jnp.dot(hblk, wc_ref[...], preferred_element_type=F32)


def _precompute(hp, W_r, W_c, npad):
    bh = npad // 8
    grid = npad // bh
    return pl.pallas_call(
        _pre_body,
        grid=(grid,),
        in_specs=[
            pl.BlockSpec((bh, 128), lambda i: (i, 0)),
            pl.BlockSpec((128, 128), lambda i: (0, 0)),
            pl.BlockSpec((128, 128), lambda i: (0, 0)),
        ],
        out_specs=[
            pl.BlockSpec((bh, 128), lambda i: (i, 0)),
            pl.BlockSpec((bh, 128), lambda i: (i, 0)),
        ],
        out_shape=[
            jax.ShapeDtypeStruct((npad, 128), F32),
            jax.ShapeDtypeStruct((npad, 128), F32),
        ],
    )(hp, W_r, W_c)


def _make_gather(epad, epw):
    nchunk = epw // CHUNK
    mesh = plsc.VectorSubcoreMesh(
        core_axis_name="c", subcore_axis_name="s", num_cores=NC, num_subcores=NS)

    @functools.partial(
        pl.kernel,
        out_type=(
            jax.ShapeDtypeStruct((epad, 128), F32),
            jax.ShapeDtypeStruct((epad, 16), F32),
            jax.ShapeDtypeStruct((epad, 16), F32),
        ),
        mesh=mesh,
        scratch_types=[
            pltpu.VMEM((CHUNK,), I32),
            pltpu.VMEM((CHUNK,), I32),
            pltpu.VMEM((CHUNK, 128), F32),
            pltpu.VMEM((CHUNK, 128), F32),
            pltpu.VMEM((CHUNK, 16), F32),
            pltpu.VMEM((CHUNK, 16), F32),
            pltpu.SemaphoreType.DMA,
            pltpu.SemaphoreType.DMA,
            pltpu.SemaphoreType.DMA,
            pltpu.SemaphoreType.DMA,
        ],
        compiler_params=pltpu.CompilerParams(use_tc_tiling_on_sc=False),
    )
    def gather_kernel(row_h, col_h, pr_h, pc_h, x16_h, pre_h, xr_h, xc_h,
                      idx_r, idx_c, buf_a, buf_b, xb_a, xb_b, s1, s2, s3, s4):
        wid = lax.axis_index("s") * NC + lax.axis_index("c")
        base0 = wid * epw

        def chunk(t, carry):
            base = base0 + t * CHUNK
            pltpu.sync_copy(row_h.at[pl.ds(base, CHUNK)], idx_r)
            pltpu.sync_copy(col_h.at[pl.ds(base, CHUNK)], idx_c)
            cp1 = pltpu.async_copy(pr_h.at[idx_r], buf_a, s1)
            cp2 = pltpu.async_copy(pc_h.at[idx_c], buf_b, s2)
            cp3 = pltpu.async_copy(x16_h.at[idx_r], xb_a, s3)
            cp4 = pltpu.async_copy(x16_h.at[idx_c], xb_b, s4)
            cp1.wait()
            cp2.wait()
            cp3.wait()
            cp4.wait()

            def addrow(i, c2):
                for j in range(8):
                    sl = pl.ds(j * 16, 16)
                    buf_a[i, sl] = buf_a[i, sl] + buf_b[i, sl]
                return c2

            lax.fori_loop(0, CHUNK, addrow, 0)
            pltpu.sync_copy(buf_a, pre_h.at[pl.ds(base, CHUNK)])
            pltpu.sync_copy(xb_a, xr_h.at[pl.ds(base, CHUNK)])
            pltpu.sync_copy(xb_b, xc_h.at[pl.ds(base, CHUNK)])
            return carry

        lax.fori_loop(0, nchunk, chunk, 0)

    return gather_kernel


def _edge_body(pre_ref, dist_ref, xr_ref, xc_ref, wd_ref, be1_ref, we2_ref,
               be2_ref, wa_ref, ba_ref, wc1_ref, bc1_ref, wc2_ref,
               msg_ref, msg2_ref, crd_ref):
    t = pre_ref[...] + dist_ref[...] * wd_ref[...] + be1_ref[...]
    t = _silu(t)
    m = jnp.dot(t, we2_ref[...], preferred_element_type=F32) + be2_ref[...]
    a = jax.nn.sigmoid(jnp.dot(m, wa_ref[...], preferred_element_type=F32)
                       + ba_ref[...])
    msg = m * a
    msg_ref[...] = msg[:, :64]
    msg2_ref[...] = msg[:, 64:]
    c = _silu(jnp.dot(msg, wc1_ref[...], preferred_element_type=F32)
              + bc1_ref[...])
    cw = jnp.dot(c, wc2_ref[...], preferred_element_type=F32)
    crd_ref[...] = (xr_ref[...] - xc_ref[...]) * cw


def _edge_mlp(pre, dist2, xr, xc, wd, be1, We2, be2, Wa, ba, Wc1, bc1, Wc2,
              epad):
    be = 4096
    grid = epad // be
    full = lambda i: (0, 0)
    return pl.pallas_call(
        _edge_body,
        grid=(grid,),
        in_specs=[
            pl.BlockSpec((be, 128), lambda i: (i, 0)),
            pl.BlockSpec((be, 1), lambda i: (i, 0)),
            pl.BlockSpec((be, 16), lambda i: (i, 0)),
            pl.BlockSpec((be, 16), lambda i: (i, 0)),
            pl.BlockSpec((1, 128), full),
            pl.BlockSpec((1, 128), full),
            pl.BlockSpec((128, 128), full),
            pl.BlockSpec((1, 128), full),
            pl.BlockSpec((128, 1), full),
            pl.BlockSpec((1, 1), full),
            pl.BlockSpec((128, 128), full),
            pl.BlockSpec((1, 128), full),
            pl.BlockSpec((128, 1), full),
        ],
        out_specs=[
            pl.BlockSpec((be, 64), lambda i: (i, 0)),
            pl.BlockSpec((be, 64), lambda i: (i, 0)),
            pl.BlockSpec((be, 16), lambda i: (i, 0)),
        ],
        out_shape=[
            jax.ShapeDtypeStruct((epad, 64), F32),
            jax.ShapeDtypeStruct((epad, 64), F32),
            jax.ShapeDtypeStruct((epad, 16), F32),
        ],
    )(pre, dist2, xr, xc, wd, be1, We2, be2, Wa, ba, Wc1, bc1, Wc2)


def _make_scatter(epad, npad):
    ept = epad // NS
    nchunk = ept // CHUNK
    rpt = npad // NS
    mesh = plsc.VectorSubcoreMesh(
        core_axis_name="c", subcore_axis_name="s", num_cores=NC, num_subcores=NS)

    @functools.partial(
        pl.kernel,
        out_type=(
            jax.ShapeDtypeStruct((npad, 64), F32),
            jax.ShapeDtypeStruct((npad, 64), F32),
            jax.ShapeDtypeStruct((npad, 16), F32),
        ),
        mesh=mesh,
        scratch_types=[
            pltpu.VMEM((CHUNK,), I32),
            pltpu.VMEM((CHUNK, 64), F32),
            pltpu.VMEM((CHUNK, 16), F32),
            pltpu.VMEM((rpt, 64), F32),
            pltpu.VMEM((rpt, 16), F32),
            pltpu.VMEM_SHARED((npad, 64), F32),
            pltpu.VMEM_SHARED((npad, 16), F32),
        ],
        compiler_params=pltpu.CompilerParams(use_tc_tiling_on_sc=False),
    )
    def scatter_kernel(col_h, msga_h, msgb_h, crd_h, agga_h, aggb_h, xd_h,
                       idx, mbuf, cbuf, z64, z16, agg_sh, xd_sh):
        cid = lax.axis_index("c")
        sid = lax.axis_index("s")
        zv = jnp.zeros((16,), F32)

        def zrow(i, carry):
            for j in range(4):
                z64[i, pl.ds(j * 16, 16)] = zv
            z16[i, pl.ds(0, 16)] = zv
            return carry

        lax.fori_loop(0, rpt, zrow, 0)
        pltpu.sync_copy(z64, agg_sh.at[pl.ds(sid * rpt, rpt)])
        pltpu.sync_copy(z16, xd_sh.at[pl.ds(sid * rpt, rpt)])
        plsc.subcore_barrier()

        base0 = sid * ept

        def chunk_a(t, carry):
            base = base0 + t * CHUNK
            pltpu.sync_copy(col_h.at[pl.ds(base, CHUNK)], idx)
            pltpu.sync_copy(msga_h.at[pl.ds(base, CHUNK)], mbuf)
            pltpu.sync_copy(mbuf, agg_sh.at[idx], add=True)
            return carry

        def chunk_b(t, carry):
            base = base0 + t * CHUNK
            pltpu.sync_copy(col_h.at[pl.ds(base, CHUNK)], idx)
            pltpu.sync_copy(msgb_h.at[pl.ds(base, CHUNK)], mbuf)
            pltpu.sync_copy(crd_h.at[pl.ds(base, CHUNK)], cbuf)
            pltpu.sync_copy(mbuf, agg_sh.at[idx], add=True)
            pltpu.sync_copy(cbuf, xd_sh.at[idx], add=True)
            return carry

        @pl.when(cid == 0)
        def _():
            lax.fori_loop(0, nchunk, chunk_a, 0)

        @pl.when(cid == 1)
        def _():
            lax.fori_loop(0, nchunk, chunk_b, 0)

        plsc.subcore_barrier()
        rows = pl.ds(sid * rpt, rpt)

        @pl.when(cid == 0)
        def _():
            pltpu.sync_copy(agg_sh.at[rows], agga_h.at[rows])

        @pl.when(cid == 1)
        def _():
            pltpu.sync_copy(agg_sh.at[rows], aggb_h.at[rows])
            pltpu.sync_copy(xd_sh.at[rows], xd_h.at[rows])

    return scatter_kernel


def _node_body(h_ref, x16_ref, agga_ref, aggb_ref, xd_ref, wn1h_ref, wn1a_ref,
               bn1_ref, wn2_ref, bn2_ref, g_ref, b_ref, hnew_ref, xnew_ref):
    h = h_ref[...]
    agg = jnp.concatenate([agga_ref[...], aggb_ref[...]], axis=-1)
    t = (jnp.dot(h, wn1h_ref[...], preferred_element_type=F32)
         + jnp.dot(agg, wn1a_ref[...], preferred_element_type=F32)
         + bn1_ref[...])
    t = _silu(t)
    mid = jnp.dot(t, wn2_ref[...], preferred_element_type=F32) + bn2_ref[...]
    y = h + mid
    mu = jnp.mean(y, axis=-1, keepdims=True)
    var = jnp.mean((y - mu) ** 2, axis=-1, keepdims=True)
    hnew_ref[...] = (y - mu) / jnp.sqrt(var + 1e-5) * g_ref[...] + b_ref[...]
    xnew_ref[...] = x16_ref[...] + xd_ref[...]


def _node_mlp(h, x16, agga, aggb, xd, Wn1h, Wn1a, bn1, Wn2, bn2, g, b, n):
    bn = 2000
    grid = n // bn
    full = lambda i: (0, 0)
    return pl.pallas_call(
        _node_body,
        grid=(grid,),
        in_specs=[
            pl.BlockSpec((bn, 128), lambda i: (i, 0)),
            pl.BlockSpec((bn, 16), lambda i: (i, 0)),
            pl.BlockSpec((bn, 64), lambda i: (i, 0)),
            pl.BlockSpec((bn, 64), lambda i: (i, 0)),
            pl.BlockSpec((bn, 16), lambda i: (i, 0)),
            pl.BlockSpec((128, 128), full),
            pl.BlockSpec((128, 128), full),
            pl.BlockSpec((1, 128), full),
            pl.BlockSpec((128, 128), full),
            pl.BlockSpec((1, 128), full),
            pl.BlockSpec((1, 128), full),
            pl.BlockSpec((1, 128), full),
        ],
        out_specs=[
            pl.BlockSpec((bn, 128), lambda i: (i, 0)),
            pl.BlockSpec((bn, 16), lambda i: (i, 0)),
        ],
        out_shape=[
            jax.ShapeDtypeStruct((n, 128), F32),
            jax.ShapeDtypeStruct((n, 16), F32),
        ],
    )(h, x16, agga, aggb, xd, Wn1h, Wn1a, bn1, Wn2, bn2, g, b)


def kernel(h, x, edge_index, edge_dist, W_e1, b_e1, W_e2, b_e2, W_a, b_a,
           W_n1, b_n1, W_n2, b_n2, W_c1, b_c1, W_c2, ln_g, ln_b):
    n, d = h.shape
    e = edge_index.shape[1]
    assert d == 128

    npad = ((n + 1 + 127) // 128) * 128
    epad = ((e + NW * CHUNK - 1) // (NW * CHUNK)) * (NW * CHUNK)
    epw = epad // NW

    row = edge_index[0].astype(I32)
    col = edge_index[1].astype(I32)
    pad_e = epad - e
    row_p = jnp.concatenate([row, jnp.full((pad_e,), n, I32)])
    col_p = jnp.concatenate([col, jnp.full((pad_e,), n, I32)])
    dist2 = jnp.concatenate([edge_dist, jnp.zeros((pad_e,), F32)])[:, None]

    hp = jnp.zeros((npad, 128), F32).at[:n].set(h)
    x16 = jnp.zeros((npad, 16), F32).at[:n, :3].set(x)

    W_r = W_e1[:128]
    W_c = W_e1[128:256]
    w_d = W_e1[256:257]

    P_r, P_c = _precompute(hp, W_r, W_c, npad)

    pre, xr, xc = _make_gather(epad, epw)(row_p, col_p, P_r, P_c, x16)

    msga, msgb, crd = _edge_mlp(
        pre, dist2, xr, xc, w_d, b_e1[None, :], W_e2, b_e2[None, :],
        W_a, b_a[None, :], W_c1, b_c1[None, :], W_c2, epad)

    agga, aggb, xd = _make_scatter(epad, npad)(col_p, msga, msgb, crd)

    h_new, x_new16 = _node_mlp(
        h, x16[:n], agga[:n], aggb[:n], xd[:n], W_n1[:128], W_n1[128:],
        b_n1[None, :], W_n2, b_n2[None, :], ln_g[None, :], ln_b[None, :], n)

    return (h_new, x_new16[:, :3])


# speedup vs baseline: 3.0101x; 1.1821x over previous
import functools

import jax
import jax.numpy as jnp
from jax import lax
from jax.experimental import pallas as pl
from jax.experimental.pallas import tpu as pltpu
from jax.experimental.pallas import tpu_sc as plsc

F32 = jnp.float32
I32 = jnp.int32

NC = 2
NS = 16
NW = NC * NS
CHUNK = 128


def _silu(v):
    return v * jax.nn.sigmoid(v)


def _mesh():
    return plsc.VectorSubcoreMesh(
        core_axis_name="c", subcore_axis_name="s", num_cores=NC, num_subcores=NS)


def _pre_body(hp_ref, wr_ref, wc_ref, pr_ref, pc_ref):
    hblk = hp_ref[...]
    pr_ref[...] = jnp.dot(hblk, wr_ref[...], preferred_element_type=F32)
    pc_ref[...] = jnp.dot(hblk, wc_ref[...], preferred_element_type=F32)


def _precompute(hp, W_r, W_c, npad):
    bh = npad // 8
    return pl.pallas_call(
        _pre_body,
        grid=(8,),
        in_specs=[
            pl.BlockSpec((bh, 128), lambda i: (i, 0)),
            pl.BlockSpec((128, 128), lambda i: (0, 0)),
            pl.BlockSpec((128, 128), lambda i: (0, 0)),
        ],
        out_specs=[
            pl.BlockSpec((bh, 128), lambda i: (i, 0)),
            pl.BlockSpec((bh, 128), lambda i: (i, 0)),
        ],
        out_shape=[
            jax.ShapeDtypeStruct((npad, 128), F32),
            jax.ShapeDtypeStruct((npad, 128), F32),
        ],
    )(hp, W_r, W_c)


def _make_gather(epad, epw):
    nchunk = epw // CHUNK

    @functools.partial(
        pl.kernel,
        out_type=jax.ShapeDtypeStruct((epad, 128), F32),
        mesh=_mesh(),
        scratch_types=[
            pltpu.VMEM((CHUNK,), I32),
            pltpu.VMEM((CHUNK,), I32),
            pltpu.VMEM((CHUNK,), F32),
            pltpu.VMEM((128,), F32),
            pltpu.VMEM((CHUNK, 128), F32),
            pltpu.VMEM((CHUNK, 128), F32),
            pltpu.SemaphoreType.DMA,
            pltpu.SemaphoreType.DMA,
        ],
        compiler_params=pltpu.CompilerParams(use_tc_tiling_on_sc=False),
    )
    def gather_kernel(row_h, col_h, dist_h, pr_h, pc_h, wd_h, pre_h,
                      idx_r, idx_c, db, wdv, buf_a, buf_b, s1, s2):
        wid = lax.axis_index("s") * NC + lax.axis_index("c")
        base0 = wid * epw
        pltpu.sync_copy(wd_h, wdv)
        wds = [wdv[pl.ds(j * 16, 16)] for j in range(8)]

        def chunk(t, carry):
            base = base0 + t * CHUNK
            pltpu.sync_copy(row_h.at[pl.ds(base, CHUNK)], idx_r)
            pltpu.sync_copy(col_h.at[pl.ds(base, CHUNK)], idx_c)
            pltpu.sync_copy(dist_h.at[pl.ds(base, CHUNK)], db)
            cp1 = pltpu.async_copy(pr_h.at[idx_r], buf_a, s1)
            cp2 = pltpu.async_copy(pc_h.at[idx_c], buf_b, s2)
            cp1.wait()
            cp2.wait()

            def addgrp(g, c2):
                dv = db[pl.ds(g * 16, 16)]
                for r in range(16):
                    i = g * 16 + r
                    d = dv[r]
                    for j in range(8):
                        sl = pl.ds(j * 16, 16)
                        buf_a[i, sl] = buf_a[i, sl] + buf_b[i, sl] + d * wds[j]
                return c2

            lax.fori_loop(0, CHUNK // 16, addgrp, 0)
            pltpu.sync_copy(buf_a, pre_h.at[pl.ds(base, CHUNK)])
            return carry

        lax.fori_loop(0, nchunk, chunk, 0)

    return gather_kernel


def _edge_body(pre_ref, be1_ref, we2_ref, be2_ref, wa_ref, ba_ref,
               wc1_ref, bc1_ref, wc2t_ref, msg_ref, cwt_ref):
    t = _silu(pre_ref[...] + be1_ref[...])
    m = jnp.dot(t, we2_ref[...], preferred_element_type=F32) + be2_ref[...]
    a = jax.nn.sigmoid(jnp.dot(m, wa_ref[...], preferred_element_type=F32)
                       + ba_ref[...])
    msg = m * a
    msg_ref[...] = msg
    c = _silu(jnp.dot(msg, wc1_ref[...], preferred_element_type=F32)
              + bc1_ref[...])
    cwt_ref[...] = jax.lax.dot_general(
        wc2t_ref[...], c, (((1,), (1,)), ((), ())),
        preferred_element_type=F32)


def _edge_mlp(pre, be1, We2, be2, Wa, ba, Wc1, bc1, Wc2t, epad):
    be = 4096
    grid = epad // be
    full = lambda i: (0, 0)
    return pl.pallas_call(
        _edge_body,
        grid=(grid,),
        in_specs=[
            pl.BlockSpec((be, 128), lambda i: (i, 0)),
            pl.BlockSpec((1, 128), full),
            pl.BlockSpec((128, 128), full),
            pl.BlockSpec((1, 128), full),
            pl.BlockSpec((128, 1), full),
            pl.BlockSpec((1, 1), full),
            pl.BlockSpec((128, 128), full),
            pl.BlockSpec((1, 128), full),
            pl.BlockSpec((1, 128), full),
        ],
        out_specs=[
            pl.BlockSpec((be, 128), lambda i: (i, 0)),
            pl.BlockSpec((1, be), lambda i: (0, i)),
        ],
        out_shape=[
            jax.ShapeDtypeStruct((epad, 128), F32),
            jax.ShapeDtypeStruct((1, epad), F32),
        ],
    )(pre, be1, We2, be2, Wa, ba, Wc1, bc1, Wc2t)


def _make_scatter(epad, npad):
    ept = epad // NS
    nchunk = ept // CHUNK
    rpt = npad // NS

    @functools.partial(
        pl.kernel,
        out_type=(
            jax.ShapeDtypeStruct((npad, 64), F32),
            jax.ShapeDtypeStruct((npad, 64), F32),
        ),
        mesh=_mesh(),
        scratch_types=[
            pltpu.VMEM((CHUNK,), I32),
            pltpu.VMEM((CHUNK, 64), F32),
            pltpu.VMEM((rpt, 64), F32),
            pltpu.VMEM_SHARED((npad, 64), F32),
        ],
        compiler_params=pltpu.CompilerParams(use_tc_tiling_on_sc=False),
    )
    def scatter_kernel(col_h, msg_h, agga_h, aggb_h, idx_c, mbuf, z64, agg_sh):
        cid = lax.axis_index("c")
        sid = lax.axis_index("s")
        zv = jnp.zeros((16,), F32)

        def zrow(i, carry):
            for j in range(4):
                z64[i, pl.ds(j * 16, 16)] = zv
            return carry

        lax.fori_loop(0, rpt, zrow, 0)
        pltpu.sync_copy(z64, agg_sh.at[pl.ds(sid * rpt, rpt)])
        plsc.subcore_barrier()

        base0 = sid * ept

        def chunk(t, carry):
            base = base0 + t * CHUNK
            pltpu.sync_copy(col_h.at[pl.ds(base, CHUNK)], idx_c)
            pltpu.sync_copy(
                msg_h.at[pl.ds(base, CHUNK), pl.ds(cid * 64, 64)], mbuf)
            pltpu.sync_copy(mbuf, agg_sh.at[idx_c], add=True)
            return carry

        lax.fori_loop(0, nchunk, chunk, 0)
        plsc.subcore_barrier()
        rows = pl.ds(sid * rpt, rpt)

        @pl.when(cid == 0)
        def _():
            pltpu.sync_copy(agg_sh.at[rows], agga_h.at[rows])

        @pl.when(cid == 1)
        def _():
            pltpu.sync_copy(agg_sh.at[rows], aggb_h.at[rows])

    return scatter_kernel


def _make_coord(epad, npad):
    ept = epad // NS
    nchunk = ept // CHUNK
    half = nchunk // 2
    rpt = npad // NS

    @functools.partial(
        pl.kernel,
        out_type=(
            jax.ShapeDtypeStruct((npad, 16), F32),
            jax.ShapeDtypeStruct((npad, 16), F32),
        ),
        mesh=_mesh(),
        scratch_types=[
            pltpu.VMEM((CHUNK,), I32),
            pltpu.VMEM((CHUNK,), I32),
            pltpu.VMEM((CHUNK,), F32),
            pltpu.VMEM((CHUNK, 16), F32),
            pltpu.VMEM((rpt, 16), F32),
            pltpu.VMEM((npad,), F32),
            pltpu.VMEM((npad,), F32),
            pltpu.VMEM((npad,), F32),
            pltpu.VMEM_SHARED((npad, 16), F32),
        ],
        compiler_params=pltpu.CompilerParams(use_tc_tiling_on_sc=False,
                                             needs_layout_passes=False),
    )
    def coord_kernel(row_h, col_h, cw_h, x3_h, xda_h, xdb_h,
                     idx_c, idx_r, cwb, cbuf, z16, xv0, xv1, xv2, xd_sh):
        cid = lax.axis_index("c")
        sid = lax.axis_index("s")
        zv = jnp.zeros((16,), F32)

        def zrow(i, carry):
            z16[i, pl.ds(0, 16)] = zv
            return carry

        lax.fori_loop(0, rpt, zrow, 0)

        def zcb(i, carry):
            cbuf[i, pl.ds(0, 16)] = zv
            return carry

        lax.fori_loop(0, CHUNK, zcb, 0)
        pltpu.sync_copy(x3_h.at[0], xv0)
        pltpu.sync_copy(x3_h.at[1], xv1)
        pltpu.sync_copy(x3_h.at[2], xv2)
        pltpu.sync_copy(z16, xd_sh.at[pl.ds(sid * rpt, rpt)])
        plsc.subcore_barrier()

        base0 = sid * ept + cid * half * CHUNK
        xvs = (xv0, xv1, xv2)

        def chunk(t, carry):
            base = base0 + t * CHUNK
            pltpu.sync_copy(col_h.at[pl.ds(base, CHUNK)], idx_c)
            pltpu.sync_copy(row_h.at[pl.ds(base, CHUNK)], idx_r)
            pltpu.sync_copy(cw_h.at[pl.ds(base, CHUNK)], cwb)
            for g in range(8):
                sl = pl.ds(g * 16, 16)
                iv_r = idx_r[sl]
                iv_c = idx_c[sl]
                cwv = cwb[sl]
                rows = lax.iota(I32, 16) + g * 16
                for k in range(3):
                    xr = plsc.load_gather(xvs[k], [iv_r])
                    xc = plsc.load_gather(xvs[k], [iv_c])
                    cols = jnp.full((16,), k, I32)
                    plsc.store_scatter(cbuf, [rows, cols], (xr - xc) * cwv)
            pltpu.sync_copy(cbuf, xd_sh.at[idx_c], add=True)
            return carry

        lax.fori_loop(0, half, chunk, 0)
        plsc.subcore_barrier()
        rows = pl.ds(sid * rpt, rpt)

        @pl.when(cid == 0)
        def _():
            pltpu.sync_copy(xd_sh.at[rows], xda_h.at[rows])

        @pl.when(cid == 1)
        def _():
            pltpu.sync_copy(xd_sh.at[rows], xdb_h.at[rows])

    return coord_kernel


def _node_body(h_ref, x16_ref, agga_ref, aggb_ref, xda_ref, xdb_ref,
               wn1h_ref, wn1a_ref, bn1_ref, wn2_ref, bn2_ref, g_ref, b_ref,
               hnew_ref, xnew_ref):
    h = h_ref[...]
    agg = jnp.concatenate([agga_ref[...], aggb_ref[...]], axis=-1)
    t = (jnp.dot(h, wn1h_ref[...], preferred_element_type=F32)
         + jnp.dot(agg, wn1a_ref[...], preferred_element_type=F32)
         + bn1_ref[...])
    t = _silu(t)
    mid = jnp.dot(t, wn2_ref[...], preferred_element_type=F32) + bn2_ref[...]
    y = h + mid
    mu = jnp.mean(y, axis=-1, keepdims=True)
    var = jnp.mean((y - mu) ** 2, axis=-1, keepdims=True)
    hnew_ref[...] = (y - mu) / jnp.sqrt(var + 1e-5) * g_ref[...] + b_ref[...]
    xnew_ref[...] = x16_ref[...] + xda_ref[...] + xdb_ref[...]


def _node_mlp(h, x16, agga, aggb, xda, xdb, Wn1h, Wn1a, bn1, Wn2, bn2, g, b, n):
    bn = 2000
    grid = n // bn
    full = lambda i: (0, 0)
    return pl.pallas_call(
        _node_body,
        grid=(grid,),
        in_specs=[
            pl.BlockSpec((bn, 128), lambda i: (i, 0)),
            pl.BlockSpec((bn, 16), lambda i: (i, 0)),
            pl.BlockSpec((bn, 64), lambda i: (i, 0)),
            pl.BlockSpec((bn, 64), lambda i: (i, 0)),
            pl.BlockSpec((bn, 16), lambda i: (i, 0)),
            pl.BlockSpec((bn, 16), lambda i: (i, 0)),
            pl.BlockSpec((128, 128), full),
            pl.BlockSpec((128, 128), full),
            pl.BlockSpec((1, 128), full),
            pl.BlockSpec((128, 128), full),
            pl.BlockSpec((1, 128), full),
            pl.BlockSpec((1, 128), full),
            pl.BlockSpec((1, 128), full),
        ],
        out_specs=[
            pl.BlockSpec((bn, 128), lambda i: (i, 0)),
            pl.BlockSpec((bn, 16), lambda i: (i, 0)),
        ],
        out_shape=[
            jax.ShapeDtypeStruct((n, 128), F32),
            jax.ShapeDtypeStruct((n, 16), F32),
        ],
    )(h, x16, agga, aggb, xda, xdb, Wn1h, Wn1a, bn1, Wn2, bn2, g, b)


def kernel(h, x, edge_index, edge_dist, W_e1, b_e1, W_e2, b_e2, W_a, b_a,
           W_n1, b_n1, W_n2, b_n2, W_c1, b_c1, W_c2, ln_g, ln_b):
    n, d = h.shape
    e = edge_index.shape[1]
    assert d == 128

    npad = ((n + 1 + 127) // 128) * 128
    epad = ((e + 2 * NW * CHUNK - 1) // (2 * NW * CHUNK)) * (2 * NW * CHUNK)
    epw = epad // NW

    row = edge_index[0].astype(I32)
    col = edge_index[1].astype(I32)
    pad_e = epad - e
    row_p = jnp.concatenate([row, jnp.full((pad_e,), n, I32)])
    col_p = jnp.concatenate([col, jnp.full((pad_e,), n, I32)])
    dist_p = jnp.concatenate([edge_dist, jnp.zeros((pad_e,), F32)])

    hp = jnp.zeros((npad, 128), F32).at[:n].set(h)
    x16 = jnp.zeros((n, 16), F32).at[:, :3].set(x)
    x3 = jnp.zeros((3, npad), F32).at[:, :n].set(x.T)

    W_r = W_e1[:128]
    W_c = W_e1[128:256]
    w_d = W_e1[256]

    P_r, P_c = _precompute(hp, W_r, W_c, npad)

    pre = _make_gather(epad, epw)(row_p, col_p, dist_p, P_r, P_c, w_d)

    msg, cwt = _edge_mlp(
        pre, b_e1[None, :], W_e2, b_e2[None, :], W_a, b_a[None, :],
        W_c1, b_c1[None, :], W_c2.reshape(1, 128), epad)

    agga, aggb = _make_scatter(epad, npad)(col_p, msg)
    xda, xdb = _make_coord(epad, npad)(row_p, col_p, cwt[0], x3)

    h_new, x_new16 = _node_mlp(
        h, x16, agga[:n], aggb[:n], xda[:n], xdb[:n], W_n1[:128], W_n1[128:],
        b_n1[None, :], W_n2, b_n2[None, :], ln_g[None, :], ln_b[None, :], n)

    return (h_new, x_new16[:, :3])


# speedup vs baseline: 3.8709x; 1.2860x over previous
import functools

import jax
import jax.numpy as jnp
from jax import lax
from jax.experimental import pallas as pl
from jax.experimental.pallas import tpu as pltpu
from jax.experimental.pallas import tpu_sc as plsc

F32 = jnp.float32
I32 = jnp.int32

NC = 2
NS = 16
NW = NC * NS
CHUNK = 128


def _silu(v):
    return v * jax.nn.sigmoid(v)


def _mesh():
    return plsc.VectorSubcoreMesh(
        core_axis_name="c", subcore_axis_name="s", num_cores=NC, num_subcores=NS)


def _pre_body(hp_ref, wr_ref, wc_ref, pr_ref, pc_ref):
    hblk = hp_ref[...]
    pr_ref[...] = jnp.dot(hblk, wr_ref[...], preferred_element_type=F32)
    pc_ref[...] = jnp.dot(hblk, wc_ref[...], preferred_element_type=F32)


def _precompute(hp, W_r, W_c, npad):
    bh = npad // 8
    return pl.pallas_call(
        _pre_body,
        grid=(8,),
        in_specs=[
            pl.BlockSpec((bh, 128), lambda i: (i, 0)),
            pl.BlockSpec((128, 128), lambda i: (0, 0)),
            pl.BlockSpec((128, 128), lambda i: (0, 0)),
        ],
        out_specs=[
            pl.BlockSpec((bh, 128), lambda i: (i, 0)),
            pl.BlockSpec((bh, 128), lambda i: (i, 0)),
        ],
        out_shape=[
            jax.ShapeDtypeStruct((npad, 128), F32),
            jax.ShapeDtypeStruct((npad, 128), F32),
        ],
    )(hp, W_r, W_c)


def _make_gather(epad, epw):
    nchunk = epw // CHUNK
    ng = nchunk // 2
    assert nchunk % 2 == 0 and ng >= 3

    @functools.partial(
        pl.kernel,
        out_type=jax.ShapeDtypeStruct((epad, 128), F32),
        mesh=_mesh(),
        scratch_types=[
            pltpu.VMEM((CHUNK,), I32),
            pltpu.VMEM((CHUNK,), I32),
            pltpu.VMEM((CHUNK,), F32),
            pltpu.VMEM((CHUNK,), I32),
            pltpu.VMEM((CHUNK,), I32),
            pltpu.VMEM((CHUNK,), F32),
            pltpu.VMEM((128,), F32),
            pltpu.VMEM((CHUNK, 128), F32),
            pltpu.VMEM((CHUNK, 128), F32),
            pltpu.VMEM((CHUNK, 128), F32),
            pltpu.VMEM((CHUNK, 128), F32),
            pltpu.VMEM((CHUNK, 128), F32),
            pltpu.VMEM((CHUNK, 128), F32),
            pltpu.SemaphoreType.DMA,
            pltpu.SemaphoreType.DMA,
            pltpu.SemaphoreType.DMA,
            pltpu.SemaphoreType.DMA,
            pltpu.SemaphoreType.DMA,
            pltpu.SemaphoreType.DMA,
            pltpu.SemaphoreType.DMA,
            pltpu.SemaphoreType.DMA,
        ],
        compiler_params=pltpu.CompilerParams(use_tc_tiling_on_sc=False),
    )
    def gather_kernel(row_h, col_h, dist_h, pr_h, pc_h, wd_h, pre_h,
                      ir_a, ic_a, id_a, ir_b, ic_b, id_b, wdv,
                      a_r, a_c, b_r, b_c, o_a, o_b,
                      s_ia, s_ib, s_ar, s_ac, s_br, s_bc, s_wa, s_wb):
        wid = lax.axis_index("s") * NC + lax.axis_index("c")
        base0 = wid * epw
        pltpu.sync_copy(wd_h, wdv)

        def fire_rc(t, ir, ic, si):
            sl = pl.ds(base0 + t * CHUNK, CHUNK)
            pltpu.async_copy(row_h.at[sl], ir, si)
            pltpu.async_copy(col_h.at[sl], ic, si)

        def wait_rc(t, ir, ic, si):
            sl = pl.ds(base0 + t * CHUNK, CHUNK)
            pltpu.make_async_copy(row_h.at[sl], ir, si).wait()
            pltpu.make_async_copy(col_h.at[sl], ic, si).wait()

        def fire_d(t, idv, si):
            sl = pl.ds(base0 + t * CHUNK, CHUNK)
            pltpu.async_copy(dist_h.at[sl], idv, si)

        def wait_d(t, idv, si):
            sl = pl.ds(base0 + t * CHUNK, CHUNK)
            pltpu.make_async_copy(dist_h.at[sl], idv, si).wait()

        def fire_g(ir, ic, br, bc, sr, sc):
            pltpu.async_copy(pr_h.at[ir], br, sr)
            pltpu.async_copy(pc_h.at[ic], bc, sc)

        def waitg(ir, ic, br, bc, sr, sc):
            pltpu.make_async_copy(pr_h.at[ir], br, sr).wait()
            pltpu.make_async_copy(pc_h.at[ic], bc, sc).wait()

        def fma(idv, br, bc, o):
            def grp(g, c2):
                dv = idv[pl.ds(g * 16, 16)]
                for r in range(16):
                    i = g * 16 + r
                    d = dv[r]
                    for j in range(8):
                        sl = pl.ds(j * 16, 16)
                        o[i, sl] = br[i, sl] + bc[i, sl] + d * wdv[sl]
                return c2

            lax.fori_loop(0, CHUNK // 16, grp, 0)

        def firew(t, o, sw):
            pltpu.async_copy(o, pre_h.at[pl.ds(base0 + t * CHUNK, CHUNK)], sw)

        def waitw(t, o, sw):
            pltpu.make_async_copy(
                o, pre_h.at[pl.ds(base0 + t * CHUNK, CHUNK)], sw).wait()

        seta = (ir_a, ic_a, id_a, s_ia, a_r, a_c, s_ar, s_ac)
        setb = (ir_b, ic_b, id_b, s_ib, b_r, b_c, s_br, s_bc)

        def step(t, st, o, sw, first, last):
            ir, ic, idv, si, br, bc, sr, sc = st
            waitg(ir, ic, br, bc, sr, sc)
            if not last:
                fire_rc(t + 2, ir, ic, si)
            if not first:
                waitw(t - 2, o, sw)
            wait_d(t, idv, si)
            fma(idv, br, bc, o)
            if not last:
                wait_rc(t + 2, ir, ic, si)
                fire_g(ir, ic, br, bc, sr, sc)
                fire_d(t + 2, idv, si)
            firew(t, o, sw)

        fire_rc(0, ir_a, ic_a, s_ia)
        fire_d(0, id_a, s_ia)
        fire_rc(1, ir_b, ic_b, s_ib)
        fire_d(1, id_b, s_ib)
        wait_rc(0, ir_a, ic_a, s_ia)
        fire_g(ir_a, ic_a, a_r, a_c, s_ar, s_ac)
        wait_rc(1, ir_b, ic_b, s_ib)
        fire_g(ir_b, ic_b, b_r, b_c, s_br, s_bc)
        step(0, seta, o_a, s_wa, True, False)
        step(1, setb, o_b, s_wb, True, False)

        def body(g, carry):
            step(2 * g, seta, o_a, s_wa, False, False)
            step(2 * g + 1, setb, o_b, s_wb, False, False)
            return carry

        lax.fori_loop(1, ng - 1, body, 0)

        step(nchunk - 2, seta, o_a, s_wa, False, True)
        step(nchunk - 1, setb, o_b, s_wb, False, True)
        waitw(nchunk - 2, o_a, s_wa)
        waitw(nchunk - 1, o_b, s_wb)

    return gather_kernel


def _edge_body(pre_ref, be1_ref, we2_ref, be2_ref, wa_ref, ba_ref,
               wc1_ref, bc1_ref, wc2t_ref, msg_ref, cwt_ref):
    t = _silu(pre_ref[...] + be1_ref[...])
    m = jnp.dot(t, we2_ref[...], preferred_element_type=F32) + be2_ref[...]
    a = jax.nn.sigmoid(jnp.dot(m, wa_ref[...], preferred_element_type=F32)
                       + ba_ref[...])
    msg = m * a
    msg_ref[...] = msg
    c = _silu(jnp.dot(msg, wc1_ref[...], preferred_element_type=F32)
              + bc1_ref[...])
    cwt_ref[...] = jax.lax.dot_general(
        wc2t_ref[...], c, (((1,), (1,)), ((), ())),
        preferred_element_type=F32)


def _edge_mlp(pre, be1, We2, be2, Wa, ba, Wc1, bc1, Wc2t, epad):
    be = 4096
    grid = epad // be
    full = lambda i: (0, 0)
    return pl.pallas_call(
        _edge_body,
        grid=(grid,),
        in_specs=[
            pl.BlockSpec((be, 128), lambda i: (i, 0)),
            pl.BlockSpec((1, 128), full),
            pl.BlockSpec((128, 128), full),
            pl.BlockSpec((1, 128), full),
            pl.BlockSpec((128, 1), full),
            pl.BlockSpec((1, 1), full),
            pl.BlockSpec((128, 128), full),
            pl.BlockSpec((1, 128), full),
            pl.BlockSpec((1, 128), full),
        ],
        out_specs=[
            pl.BlockSpec((be, 128), lambda i: (i, 0)),
            pl.BlockSpec((1, be), lambda i: (0, i)),
        ],
        out_shape=[
            jax.ShapeDtypeStruct((epad, 128), F32),
            jax.ShapeDtypeStruct((1, epad), F32),
        ],
    )(pre, be1, We2, be2, Wa, ba, Wc1, bc1, Wc2t)


def _make_scatter(epad, npad):
    ept = epad // NS
    nchunk = ept // CHUNK
    rpt = npad // NS

    @functools.partial(
        pl.kernel,
        out_type=(
            jax.ShapeDtypeStruct((npad, 64), F32),
            jax.ShapeDtypeStruct((npad, 64), F32),
        ),
        mesh=_mesh(),
        scratch_types=[
            pltpu.VMEM((CHUNK,), I32),
            pltpu.VMEM((CHUNK, 64), F32),
            pltpu.VMEM((rpt, 64), F32),
            pltpu.VMEM_SHARED((npad, 64), F32),
        ],
        compiler_params=pltpu.CompilerParams(use_tc_tiling_on_sc=False),
    )
    def scatter_kernel(col_h, msg_h, agga_h, aggb_h, idx_c, mbuf, z64, agg_sh):
        cid = lax.axis_index("c")
        sid = lax.axis_index("s")
        zv = jnp.zeros((16,), F32)

        def zrow(i, carry):
            for j in range(4):
                z64[i, pl.ds(j * 16, 16)] = zv
            return carry

        lax.fori_loop(0, rpt, zrow, 0)
        pltpu.sync_copy(z64, agg_sh.at[pl.ds(sid * rpt, rpt)])
        plsc.subcore_barrier()

        base0 = sid * ept

        def chunk(t, carry):
            base = base0 + t * CHUNK
            pltpu.sync_copy(col_h.at[pl.ds(base, CHUNK)], idx_c)
            pltpu.sync_copy(
                msg_h.at[pl.ds(base, CHUNK), pl.ds(cid * 64, 64)], mbuf)
            pltpu.sync_copy(mbuf, agg_sh.at[idx_c], add=True)
            return carry

        lax.fori_loop(0, nchunk, chunk, 0)
        plsc.subcore_barrier()
        rows = pl.ds(sid * rpt, rpt)

        @pl.when(cid == 0)
        def _():
            pltpu.sync_copy(agg_sh.at[rows], agga_h.at[rows])

        @pl.when(cid == 1)
        def _():
            pltpu.sync_copy(agg_sh.at[rows], aggb_h.at[rows])

    return scatter_kernel


def _make_coord(epad, npad):
    ept = epad // NS
    nchunk = ept // CHUNK
    half = nchunk // 2
    rpt = npad // NS

    @functools.partial(
        pl.kernel,
        out_type=(
            jax.ShapeDtypeStruct((npad, 16), F32),
            jax.ShapeDtypeStruct((npad, 16), F32),
        ),
        mesh=_mesh(),
        scratch_types=[
            pltpu.VMEM((CHUNK,), I32),
            pltpu.VMEM((CHUNK,), I32),
            pltpu.VMEM((CHUNK,), F32),
            pltpu.VMEM((CHUNK, 16), F32),
            pltpu.VMEM((rpt, 16), F32),
            pltpu.VMEM((npad,), F32),
            pltpu.VMEM((npad,), F32),
            pltpu.VMEM((npad,), F32),
            pltpu.VMEM_SHARED((npad, 16), F32),
        ],
        compiler_params=pltpu.CompilerParams(use_tc_tiling_on_sc=False,
                                             needs_layout_passes=False),
    )
    def coord_kernel(row_h, col_h, cw_h, x3_h, xda_h, xdb_h,
                     idx_c, idx_r, cwb, cbuf, z16, xv0, xv1, xv2, xd_sh):
        cid = lax.axis_index("c")
        sid = lax.axis_index("s")
        zv = jnp.zeros((16,), F32)

        def zrow(i, carry):
            z16[i, pl.ds(0, 16)] = zv
            return carry

        lax.fori_loop(0, rpt, zrow, 0)

        def zcb(i, carry):
            cbuf[i, pl.ds(0, 16)] = zv
            return carry

        lax.fori_loop(0, CHUNK, zcb, 0)
        pltpu.sync_copy(x3_h.at[0], xv0)
        pltpu.sync_copy(x3_h.at[1], xv1)
        pltpu.sync_copy(x3_h.at[2], xv2)
        pltpu.sync_copy(z16, xd_sh.at[pl.ds(sid * rpt, rpt)])
        plsc.subcore_barrier()

        base0 = sid * ept + cid * half * CHUNK
        xvs = (xv0, xv1, xv2)

        def chunk(t, carry):
            base = base0 + t * CHUNK
            pltpu.sync_copy(col_h.at[pl.ds(base, CHUNK)], idx_c)
            pltpu.sync_copy(row_h.at[pl.ds(base, CHUNK)], idx_r)
            pltpu.sync_copy(cw_h.at[pl.ds(base, CHUNK)], cwb)
            for g in range(8):
                sl = pl.ds(g * 16, 16)
                iv_r = idx_r[sl]
                iv_c = idx_c[sl]
                cwv = cwb[sl]
                rows = lax.iota(I32, 16) + g * 16
                for k in range(3):
                    xr = plsc.load_gather(xvs[k], [iv_r])
                    xc = plsc.load_gather(xvs[k], [iv_c])
                    cols = jnp.full((16,), k, I32)
                    plsc.store_scatter(cbuf, [rows, cols], (xr - xc) * cwv)
            pltpu.sync_copy(cbuf, xd_sh.at[idx_c], add=True)
            return carry

        lax.fori_loop(0, half, chunk, 0)
        plsc.subcore_barrier()
        rows = pl.ds(sid * rpt, rpt)

        @pl.when(cid == 0)
        def _():
            pltpu.sync_copy(xd_sh.at[rows], xda_h.at[rows])

        @pl.when(cid == 1)
        def _():
            pltpu.sync_copy(xd_sh.at[rows], xdb_h.at[rows])

    return coord_kernel


def _node_body(h_ref, x16_ref, agga_ref, aggb_ref, xda_ref, xdb_ref,
               wn1h_ref, wn1a_ref, bn1_ref, wn2_ref, bn2_ref, g_ref, b_ref,
               hnew_ref, xnew_ref):
    h = h_ref[...]
    agg = jnp.concatenate([agga_ref[...], aggb_ref[...]], axis=-1)
    t = (jnp.dot(h, wn1h_ref[...], preferred_element_type=F32)
         + jnp.dot(agg, wn1a_ref[...], preferred_element_type=F32)
         + bn1_ref[...])
    t = _silu(t)
    mid = jnp.dot(t, wn2_ref[...], preferred_element_type=F32) + bn2_ref[...]
    y = h + mid
    mu = jnp.mean(y, axis=-1, keepdims=True)
    var = jnp.mean((y - mu) ** 2, axis=-1, keepdims=True)
    hnew_ref[...] = (y - mu) / jnp.sqrt(var + 1e-5) * g_ref[...] + b_ref[...]
    xnew_ref[...] = x16_ref[...] + xda_ref[...] + xdb_ref[...]


def _node_mlp(h, x16, agga, aggb, xda, xdb, Wn1h, Wn1a, bn1, Wn2, bn2, g, b, n):
    bn = 2000
    grid = n // bn
    full = lambda i: (0, 0)
    return pl.pallas_call(
        _node_body,
        grid=(grid,),
        in_specs=[
            pl.BlockSpec((bn, 128), lambda i: (i, 0)),
            pl.BlockSpec((bn, 16), lambda i: (i, 0)),
            pl.BlockSpec((bn, 64), lambda i: (i, 0)),
            pl.BlockSpec((bn, 64), lambda i: (i, 0)),
            pl.BlockSpec((bn, 16), lambda i: (i, 0)),
            pl.BlockSpec((bn, 16), lambda i: (i, 0)),
            pl.BlockSpec((128, 128), full),
            pl.BlockSpec((128, 128), full),
            pl.BlockSpec((1, 128), full),
            pl.BlockSpec((128, 128), full),
            pl.BlockSpec((1, 128), full),
            pl.BlockSpec((1, 128), full),
            pl.BlockSpec((1, 128), full),
        ],
        out_specs=[
            pl.BlockSpec((bn, 128), lambda i: (i, 0)),
            pl.BlockSpec((bn, 16), lambda i: (i, 0)),
        ],
        out_shape=[
            jax.ShapeDtypeStruct((n, 128), F32),
            jax.ShapeDtypeStruct((n, 16), F32),
        ],
    )(h, x16, agga, aggb, xda, xdb, Wn1h, Wn1a, bn1, Wn2, bn2, g, b)


def kernel(h, x, edge_index, edge_dist, W_e1, b_e1, W_e2, b_e2, W_a, b_a,
           W_n1, b_n1, W_n2, b_n2, W_c1, b_c1, W_c2, ln_g, ln_b):
    n, d = h.shape
    e = edge_index.shape[1]
    assert d == 128

    npad = ((n + 1 + 127) // 128) * 128
    epad = ((e + 2 * NW * CHUNK - 1) // (2 * NW * CHUNK)) * (2 * NW * CHUNK)
    epw = epad // NW

    row = edge_index[0].astype(I32)
    col = edge_index[1].astype(I32)
    pad_e = epad - e
    row_p = jnp.concatenate([row, jnp.full((pad_e,), n, I32)])
    col_p = jnp.concatenate([col, jnp.full((pad_e,), n, I32)])
    dist_p = jnp.concatenate([edge_dist, jnp.zeros((pad_e,), F32)])

    hp = jnp.zeros((npad, 128), F32).at[:n].set(h)
    x16 = jnp.zeros((n, 16), F32).at[:, :3].set(x)
    x3 = jnp.zeros((3, npad), F32).at[:, :n].set(x.T)

    W_r = W_e1[:128]
    W_c = W_e1[128:256]
    w_d = W_e1[256]

    P_r, P_c = _precompute(hp, W_r, W_c, npad)

    pre = _make_gather(epad, epw)(row_p, col_p, dist_p, P_r, P_c, w_d)

    msg, cwt = _edge_mlp(
        pre, b_e1[None, :], W_e2, b_e2[None, :], W_a, b_a[None, :],
        W_c1, b_c1[None, :], W_c2.reshape(1, 128), epad)

    agga, aggb = _make_scatter(epad, npad)(col_p, msg)
    xda, xdb = _make_coord(epad, npad)(row_p, col_p, cwt[0], x3)

    h_new, x_new16 = _node_mlp(
        h, x16, agga[:n], aggb[:n], xda[:n], xdb[:n], W_n1[:128], W_n1[128:],
        b_n1[None, :], W_n2, b_n2[None, :], ln_g[None, :], ln_b[None, :], n)

    return (h_new, x_new16[:, :3])


# speedup vs baseline: 4.4332x; 1.1453x over previous
import functools

import jax
import jax.numpy as jnp
from jax import lax
from jax.experimental import pallas as pl
from jax.experimental.pallas import tpu as pltpu
from jax.experimental.pallas import tpu_sc as plsc

F32 = jnp.float32
I32 = jnp.int32

NC = 2
NS = 16
NW = NC * NS
CHUNK = 128


def _silu(v):
    return v * jax.nn.sigmoid(v)


def _mesh():
    return plsc.VectorSubcoreMesh(
        core_axis_name="c", subcore_axis_name="s", num_cores=NC, num_subcores=NS)


def _pre_body(hp_ref, wr_ref, wc_ref, pr_ref, pc_ref):
    hblk = hp_ref[...]
    pr_ref[...] = jnp.dot(hblk, wr_ref[...], preferred_element_type=F32)
    pc_ref[...] = jnp.dot(hblk, wc_ref[...], preferred_element_type=F32)


def _precompute(hp, W_r, W_c, npad):
    bh = npad // 8
    return pl.pallas_call(
        _pre_body,
        grid=(8,),
        in_specs=[
            pl.BlockSpec((bh, 128), lambda i: (i, 0)),
            pl.BlockSpec((128, 128), lambda i: (0, 0)),
            pl.BlockSpec((128, 128), lambda i: (0, 0)),
        ],
        out_specs=[
            pl.BlockSpec((bh, 128), lambda i: (i, 0)),
            pl.BlockSpec((bh, 128), lambda i: (i, 0)),
        ],
        out_shape=[
            jax.ShapeDtypeStruct((npad, 128), F32),
            jax.ShapeDtypeStruct((npad, 128), F32),
        ],
    )(hp, W_r, W_c)


def _make_gather(epad, epw):
    total = epad // CHUNK
    h0 = int(total // NS * 0.575) // 2 * 2
    h1 = total // NS - h0
    assert h0 % 2 == 0 and h1 % 2 == 0 and h0 >= 6 and h1 >= 6

    @functools.partial(
        pl.kernel,
        out_type=jax.ShapeDtypeStruct((epad, 128), F32),
        mesh=_mesh(),
        scratch_types=[
            pltpu.VMEM((CHUNK,), I32),
            pltpu.VMEM((CHUNK,), I32),
            pltpu.VMEM((CHUNK,), F32),
            pltpu.VMEM((CHUNK,), I32),
            pltpu.VMEM((CHUNK,), I32),
            pltpu.VMEM((CHUNK,), F32),
            pltpu.VMEM((128,), F32),
            pltpu.VMEM((CHUNK, 128), F32),
            pltpu.VMEM((CHUNK, 128), F32),
            pltpu.VMEM((CHUNK, 128), F32),
            pltpu.VMEM((CHUNK, 128), F32),
            pltpu.VMEM((CHUNK, 128), F32),
            pltpu.VMEM((CHUNK, 128), F32),
            pltpu.SemaphoreType.DMA,
            pltpu.SemaphoreType.DMA,
            pltpu.SemaphoreType.DMA,
            pltpu.SemaphoreType.DMA,
            pltpu.SemaphoreType.DMA,
            pltpu.SemaphoreType.DMA,
            pltpu.SemaphoreType.DMA,
            pltpu.SemaphoreType.DMA,
        ],
        compiler_params=pltpu.CompilerParams(use_tc_tiling_on_sc=False),
    )
    def gather_kernel(row_h, col_h, dist_h, pr_h, pc_h, wd_h, pre_h,
                      ir_a, ic_a, id_a, ir_b, ic_b, id_b, wdv,
                      a_r, a_c, b_r, b_c, o_a, o_b,
                      s_ia, s_ib, s_ar, s_ac, s_br, s_bc, s_wa, s_wb):
        cid = lax.axis_index("c")
        sid = lax.axis_index("s")
        nchunk = jnp.where(cid == 0, h0, h1)
        ng = nchunk // 2
        base0 = jnp.where(cid == 0, sid * h0, NS * h0 + sid * h1) * CHUNK
        pltpu.sync_copy(wd_h, wdv)

        def fire_rc(t, ir, ic, si):
            sl = pl.ds(base0 + t * CHUNK, CHUNK)
            pltpu.async_copy(row_h.at[sl], ir, si)
            pltpu.async_copy(col_h.at[sl], ic, si)

        def wait_rc(t, ir, ic, si):
            sl = pl.ds(base0 + t * CHUNK, CHUNK)
            pltpu.make_async_copy(row_h.at[sl], ir, si).wait()
            pltpu.make_async_copy(col_h.at[sl], ic, si).wait()

        def fire_d(t, idv, si):
            sl = pl.ds(base0 + t * CHUNK, CHUNK)
            pltpu.async_copy(dist_h.at[sl], idv, si)

        def wait_d(t, idv, si):
            sl = pl.ds(base0 + t * CHUNK, CHUNK)
            pltpu.make_async_copy(dist_h.at[sl], idv, si).wait()

        def fire_g(ir, ic, br, bc, sr, sc):
            pltpu.async_copy(pr_h.at[ir], br, sr)
            pltpu.async_copy(pc_h.at[ic], bc, sc)

        def waitg(ir, ic, br, bc, sr, sc):
            pltpu.make_async_copy(pr_h.at[ir], br, sr).wait()
            pltpu.make_async_copy(pc_h.at[ic], bc, sc).wait()

        def fma(idv, br, bc, o):
            def grp(g, c2):
                dv = idv[pl.ds(g * 16, 16)]
                for r in range(16):
                    i = g * 16 + r
                    d = dv[r]
                    for j in range(8):
                        sl = pl.ds(j * 16, 16)
                        o[i, sl] = br[i, sl] + bc[i, sl] + d * wdv[sl]
                return c2

            lax.fori_loop(0, CHUNK // 16, grp, 0)

        def firew(t, o, sw):
            pltpu.async_copy(o, pre_h.at[pl.ds(base0 + t * CHUNK, CHUNK)], sw)

        def waitw(t, o, sw):
            pltpu.make_async_copy(
                o, pre_h.at[pl.ds(base0 + t * CHUNK, CHUNK)], sw).wait()

        seta = (ir_a, ic_a, id_a, s_ia, a_r, a_c, s_ar, s_ac)
        setb = (ir_b, ic_b, id_b, s_ib, b_r, b_c, s_br, s_bc)

        def step(t, st, o, sw, first, last):
            ir, ic, idv, si, br, bc, sr, sc = st
            waitg(ir, ic, br, bc, sr, sc)
            if not last:
                fire_rc(t + 2, ir, ic, si)
            if not first:
                waitw(t - 2, o, sw)
            wait_d(t, idv, si)
            fma(idv, br, bc, o)
            if not last:
                wait_rc(t + 2, ir, ic, si)
                fire_g(ir, ic, br, bc, sr, sc)
                fire_d(t + 2, idv, si)
            firew(t, o, sw)

        fire_rc(0, ir_a, ic_a, s_ia)
        fire_d(0, id_a, s_ia)
        fire_rc(1, ir_b, ic_b, s_ib)
        fire_d(1, id_b, s_ib)
        wait_rc(0, ir_a, ic_a, s_ia)
        fire_g(ir_a, ic_a, a_r, a_c, s_ar, s_ac)
        wait_rc(1, ir_b, ic_b, s_ib)
        fire_g(ir_b, ic_b, b_r, b_c, s_br, s_bc)
        step(0, seta, o_a, s_wa, True, False)
        step(1, setb, o_b, s_wb, True, False)

        def body(g, carry):
            step(2 * g, seta, o_a, s_wa, False, False)
            step(2 * g + 1, setb, o_b, s_wb, False, False)
            return carry

        lax.fori_loop(1, ng - 1, body, 0)

        step(nchunk - 2, seta, o_a, s_wa, False, True)
        step(nchunk - 1, setb, o_b, s_wb, False, True)
        waitw(nchunk - 2, o_a, s_wa)
        waitw(nchunk - 1, o_b, s_wb)

    return gather_kernel


def _edge_body(pre_ref, be1_ref, we2_ref, be2_ref, wa_ref, ba_ref,
               wc1_ref, bc1_ref, wc2t_ref, msg_ref, cwt_ref):
    t = _silu(pre_ref[...] + be1_ref[...])
    m = jnp.dot(t, we2_ref[...], preferred_element_type=F32) + be2_ref[...]
    a = jax.nn.sigmoid(jnp.dot(m, wa_ref[...], preferred_element_type=F32)
                       + ba_ref[...])
    msg = m * a
    msg_ref[...] = msg
    c = _silu(jnp.dot(msg, wc1_ref[...], preferred_element_type=F32)
              + bc1_ref[...])
    cwt_ref[...] = jax.lax.dot_general(
        wc2t_ref[...], c, (((1,), (1,)), ((), ())),
        preferred_element_type=F32)


def _edge_mlp(pre, be1, We2, be2, Wa, ba, Wc1, bc1, Wc2t, epad):
    be = 4096
    grid = epad // be
    full = lambda i: (0, 0)
    return pl.pallas_call(
        _edge_body,
        grid=(grid,),
        in_specs=[
            pl.BlockSpec((be, 128), lambda i: (i, 0)),
            pl.BlockSpec((1, 128), full),
            pl.BlockSpec((128, 128), full),
            pl.BlockSpec((1, 128), full),
            pl.BlockSpec((128, 1), full),
            pl.BlockSpec((1, 1), full),
            pl.BlockSpec((128, 128), full),
            pl.BlockSpec((1, 128), full),
            pl.BlockSpec((1, 128), full),
        ],
        out_specs=[
            pl.BlockSpec((be, 128), lambda i: (i, 0)),
            pl.BlockSpec((1, be), lambda i: (0, i)),
        ],
        out_shape=[
            jax.ShapeDtypeStruct((epad, 128), F32),
            jax.ShapeDtypeStruct((1, epad), F32),
        ],
    )(pre, be1, We2, be2, Wa, ba, Wc1, bc1, Wc2t)


def _make_scatter(epad, npad):
    ept = epad // NS
    nchunk = ept // CHUNK
    rpt = npad // NS

    @functools.partial(
        pl.kernel,
        out_type=(
            jax.ShapeDtypeStruct((npad, 64), F32),
            jax.ShapeDtypeStruct((npad, 64), F32),
        ),
        mesh=_mesh(),
        scratch_types=[
            pltpu.VMEM((CHUNK,), I32),
            pltpu.VMEM((CHUNK, 64), F32),
            pltpu.VMEM((rpt, 64), F32),
            pltpu.VMEM_SHARED((npad, 64), F32),
        ],
        compiler_params=pltpu.CompilerParams(use_tc_tiling_on_sc=False),
    )
    def scatter_kernel(col_h, msg_h, agga_h, aggb_h, idx_c, mbuf, z64, agg_sh):
        cid = lax.axis_index("c")
        sid = lax.axis_index("s")
        zv = jnp.zeros((16,), F32)

        def zrow(i, carry):
            for j in range(4):
                z64[i, pl.ds(j * 16, 16)] = zv
            return carry

        lax.fori_loop(0, rpt, zrow, 0)
        pltpu.sync_copy(z64, agg_sh.at[pl.ds(sid * rpt, rpt)])
        plsc.subcore_barrier()

        base0 = sid * ept

        def chunk(t, carry):
            base = base0 + t * CHUNK
            pltpu.sync_copy(col_h.at[pl.ds(base, CHUNK)], idx_c)
            pltpu.sync_copy(
                msg_h.at[pl.ds(base, CHUNK), pl.ds(cid * 64, 64)], mbuf)
            pltpu.sync_copy(mbuf, agg_sh.at[idx_c], add=True)
            return carry

        lax.fori_loop(0, nchunk, chunk, 0)
        plsc.subcore_barrier()
        rows = pl.ds(sid * rpt, rpt)

        @pl.when(cid == 0)
        def _():
            pltpu.sync_copy(agg_sh.at[rows], agga_h.at[rows])

        @pl.when(cid == 1)
        def _():
            pltpu.sync_copy(agg_sh.at[rows], aggb_h.at[rows])

    return scatter_kernel


def _make_coord(epad, npad):
    ept = epad // NS
    nchunk = ept // CHUNK
    half = nchunk // 2
    rpt = npad // NS

    @functools.partial(
        pl.kernel,
        out_type=(
            jax.ShapeDtypeStruct((npad, 16), F32),
            jax.ShapeDtypeStruct((npad, 16), F32),
        ),
        mesh=_mesh(),
        scratch_types=[
            pltpu.VMEM((CHUNK,), I32),
            pltpu.VMEM((CHUNK,), I32),
            pltpu.VMEM((CHUNK,), F32),
            pltpu.VMEM((CHUNK, 16), F32),
            pltpu.VMEM((CHUNK,), I32),
            pltpu.VMEM((CHUNK,), I32),
            pltpu.VMEM((CHUNK,), F32),
            pltpu.VMEM((CHUNK, 16), F32),
            pltpu.VMEM((rpt, 16), F32),
            pltpu.VMEM((npad,), F32),
            pltpu.VMEM((npad,), F32),
            pltpu.VMEM((npad,), F32),
            pltpu.VMEM_SHARED((npad, 16), F32),
            pltpu.SemaphoreType.DMA,
            pltpu.SemaphoreType.DMA,
        ],
        compiler_params=pltpu.CompilerParams(use_tc_tiling_on_sc=False,
                                             needs_layout_passes=False),
    )
    def coord_kernel(row_h, col_h, cw_h, x3_h, xda_h, xdb_h,
                     ic_a, ir_a, cw_a, cb_a, ic_b, ir_b, cw_b, cb_b,
                     z16, xv0, xv1, xv2, xd_sh, s_a, s_b):
        cid = lax.axis_index("c")
        sid = lax.axis_index("s")
        zv = jnp.zeros((16,), F32)

        def zrow(i, carry):
            z16[i, pl.ds(0, 16)] = zv
            return carry

        lax.fori_loop(0, rpt, zrow, 0)

        def zcb(i, carry):
            cb_a[i, pl.ds(0, 16)] = zv
            cb_b[i, pl.ds(0, 16)] = zv
            return carry

        lax.fori_loop(0, CHUNK, zcb, 0)
        pltpu.sync_copy(x3_h.at[0], xv0)
        pltpu.sync_copy(x3_h.at[1], xv1)
        pltpu.sync_copy(x3_h.at[2], xv2)
        pltpu.sync_copy(z16, xd_sh.at[pl.ds(sid * rpt, rpt)])
        plsc.subcore_barrier()

        base0 = sid * ept + cid * half * CHUNK
        xvs = (xv0, xv1, xv2)

        def fire_i(t, ic, ir, cwv, si):
            sl = pl.ds(base0 + t * CHUNK, CHUNK)
            pltpu.async_copy(col_h.at[sl], ic, si)
            pltpu.async_copy(row_h.at[sl], ir, si)
            pltpu.async_copy(cw_h.at[sl], cwv, si)

        def wait_i(t, ic, ir, cwv, si):
            sl = pl.ds(base0 + t * CHUNK, CHUNK)
            pltpu.make_async_copy(col_h.at[sl], ic, si).wait()
            pltpu.make_async_copy(row_h.at[sl], ir, si).wait()
            pltpu.make_async_copy(cw_h.at[sl], cwv, si).wait()

        def work(t, ic, ir, cwb, cbuf, si, last):
            wait_i(t, ic, ir, cwb, si)
            for g in range(8):
                sl = pl.ds(g * 16, 16)
                iv_r = ir[sl]
                iv_c = ic[sl]
                cwv = cwb[sl]
                rows = lax.iota(I32, 16) + g * 16
                for k in range(3):
                    xr = plsc.load_gather(xvs[k], [iv_r])
                    xc = plsc.load_gather(xvs[k], [iv_c])
                    cols = jnp.full((16,), k, I32)
                    plsc.store_scatter(cbuf, [rows, cols], (xr - xc) * cwv)
            pltpu.sync_copy(cbuf, xd_sh.at[ic], add=True)
            if not last:
                fire_i(t + 2, ic, ir, cwb, si)

        fire_i(0, ic_a, ir_a, cw_a, s_a)
        fire_i(1, ic_b, ir_b, cw_b, s_b)

        def body(g, carry):
            t0 = 2 * g
            work(t0, ic_a, ir_a, cw_a, cb_a, s_a, False)
            work(t0 + 1, ic_b, ir_b, cw_b, cb_b, s_b, False)
            return carry

        lax.fori_loop(0, half // 2 - 1, body, 0)
        work(half - 2, ic_a, ir_a, cw_a, cb_a, s_a, True)
        work(half - 1, ic_b, ir_b, cw_b, cb_b, s_b, True)
        plsc.subcore_barrier()
        rows = pl.ds(sid * rpt, rpt)

        @pl.when(cid == 0)
        def _():
            pltpu.sync_copy(xd_sh.at[rows], xda_h.at[rows])

        @pl.when(cid == 1)
        def _():
            pltpu.sync_copy(xd_sh.at[rows], xdb_h.at[rows])

    return coord_kernel


def _node_h_body(h_ref, agga_ref, aggb_ref, wn1h_ref, wn1a_ref, bn1_ref,
                 wn2_ref, bn2_ref, g_ref, b_ref, hnew_ref):
    h = h_ref[...]
    agg = jnp.concatenate([agga_ref[...], aggb_ref[...]], axis=-1)
    t = (jnp.dot(h, wn1h_ref[...], preferred_element_type=F32)
         + jnp.dot(agg, wn1a_ref[...], preferred_element_type=F32)
         + bn1_ref[...])
    t = _silu(t)
    mid = jnp.dot(t, wn2_ref[...], preferred_element_type=F32) + bn2_ref[...]
    y = h + mid
    mu = jnp.mean(y, axis=-1, keepdims=True)
    var = jnp.mean((y - mu) ** 2, axis=-1, keepdims=True)
    hnew_ref[...] = (y - mu) / jnp.sqrt(var + 1e-5) * g_ref[...] + b_ref[...]


def _node_h(h, agga, aggb, Wn1h, Wn1a, bn1, Wn2, bn2, g, b, n):
    bn = 2000
    grid = n // bn
    full = lambda i: (0, 0)
    return pl.pallas_call(
        _node_h_body,
        grid=(grid,),
        in_specs=[
            pl.BlockSpec((bn, 128), lambda i: (i, 0)),
            pl.BlockSpec((bn, 64), lambda i: (i, 0)),
            pl.BlockSpec((bn, 64), lambda i: (i, 0)),
            pl.BlockSpec((128, 128), full),
            pl.BlockSpec((128, 128), full),
            pl.BlockSpec((1, 128), full),
            pl.BlockSpec((128, 128), full),
            pl.BlockSpec((1, 128), full),
            pl.BlockSpec((1, 128), full),
            pl.BlockSpec((1, 128), full),
        ],
        out_specs=pl.BlockSpec((bn, 128), lambda i: (i, 0)),
        out_shape=jax.ShapeDtypeStruct((n, 128), F32),
    )(h, agga, aggb, Wn1h, Wn1a, bn1, Wn2, bn2, g, b)


def _node_x_body(x16_ref, xda_ref, xdb_ref, xnew_ref):
    xnew_ref[...] = x16_ref[...] + xda_ref[...] + xdb_ref[...]


def _node_x(x16, xda, xdb, n):
    bn = 2000
    grid = n // bn
    return pl.pallas_call(
        _node_x_body,
        grid=(grid,),
        in_specs=[
            pl.BlockSpec((bn, 16), lambda i: (i, 0)),
            pl.BlockSpec((bn, 16), lambda i: (i, 0)),
            pl.BlockSpec((bn, 16), lambda i: (i, 0)),
        ],
        out_specs=pl.BlockSpec((bn, 16), lambda i: (i, 0)),
        out_shape=jax.ShapeDtypeStruct((n, 16), F32),
    )(x16, xda, xdb)


def kernel(h, x, edge_index, edge_dist, W_e1, b_e1, W_e2, b_e2, W_a, b_a,
           W_n1, b_n1, W_n2, b_n2, W_c1, b_c1, W_c2, ln_g, ln_b):
    n, d = h.shape
    e = edge_index.shape[1]
    assert d == 128

    npad = ((n + 1 + 127) // 128) * 128
    epad = ((e + 2 * NW * CHUNK - 1) // (2 * NW * CHUNK)) * (2 * NW * CHUNK)
    epw = epad // NW

    row = edge_index[0].astype(I32)
    col = edge_index[1].astype(I32)
    pad_e = epad - e
    row_p = jnp.concatenate([row, jnp.full((pad_e,), n, I32)])
    col_p = jnp.concatenate([col, jnp.full((pad_e,), n, I32)])
    dist_p = jnp.concatenate([edge_dist, jnp.zeros((pad_e,), F32)])

    hp = jnp.zeros((npad, 128), F32).at[:n].set(h)
    x16 = jnp.zeros((n, 16), F32).at[:, :3].set(x)
    x3 = jnp.zeros((3, npad), F32).at[:, :n].set(x.T)

    W_r = W_e1[:128]
    W_c = W_e1[128:256]
    w_d = W_e1[256]

    P_r, P_c = _precompute(hp, W_r, W_c, npad)

    pre = _make_gather(epad, epw)(row_p, col_p, dist_p, P_r, P_c, w_d)

    msg, cwt = _edge_mlp(
        pre, b_e1[None, :], W_e2, b_e2[None, :], W_a, b_a[None, :],
        W_c1, b_c1[None, :], W_c2.reshape(1, 128), epad)

    agga, aggb = _make_scatter(epad, npad)(col_p, msg)
    xda, xdb = _make_coord(epad, npad)(row_p, col_p, cwt[0], x3)

    h_new = _node_h(
        h, agga[:n], aggb[:n], W_n1[:128], W_n1[128:], b_n1[None, :],
        W_n2, b_n2[None, :], ln_g[None, :], ln_b[None, :], n)
    x_new16 = _node_x(x16, xda[:n], xdb[:n], n)

    return (h_new, x_new16[:, :3])


# speedup vs baseline: 5.1629x; 1.1646x over previous
import functools

import jax
import jax.numpy as jnp
from jax import lax
from jax.experimental import pallas as pl
from jax.experimental.pallas import tpu as pltpu
from jax.experimental.pallas import tpu_sc as plsc

F32 = jnp.float32
I32 = jnp.int32

NC = 2
NS = 16
NW = NC * NS
CHUNK = 128


def _silu(v):
    return v * jax.nn.sigmoid(v)


def _mesh():
    return plsc.VectorSubcoreMesh(
        core_axis_name="c", subcore_axis_name="s", num_cores=NC, num_subcores=NS)


def _pre_body(hp_ref, wr_ref, wc_ref, pr_ref, pc_ref):
    hblk = hp_ref[...]
    pr_ref[...] = jnp.dot(hblk, wr_ref[...], preferred_element_type=F32)
    pc_ref[...] = jnp.dot(hblk, wc_ref[...], preferred_element_type=F32)


def _precompute(hp, W_r, W_c, npad):
    bh = npad // 8
    return pl.pallas_call(
        _pre_body,
        grid=(8,),
        in_specs=[
            pl.BlockSpec((bh, 128), lambda i: (i, 0)),
            pl.BlockSpec((128, 128), lambda i: (0, 0)),
            pl.BlockSpec((128, 128), lambda i: (0, 0)),
        ],
        out_specs=[
            pl.BlockSpec((bh, 128), lambda i: (i, 0)),
            pl.BlockSpec((bh, 128), lambda i: (i, 0)),
        ],
        out_shape=[
            jax.ShapeDtypeStruct((npad, 128), F32),
            jax.ShapeDtypeStruct((npad, 128), F32),
        ],
    )(hp, W_r, W_c)


def _make_gather(epad, epw):
    total = epad // CHUNK
    h0 = int(total // NS * 0.575) // 2 * 2
    h1 = total // NS - h0
    assert h0 % 2 == 0 and h1 % 2 == 0 and h0 >= 6 and h1 >= 6

    @functools.partial(
        pl.kernel,
        out_type=jax.ShapeDtypeStruct((epad, 128), F32),
        mesh=_mesh(),
        scratch_types=[
            pltpu.VMEM((CHUNK,), I32),
            pltpu.VMEM((CHUNK,), I32),
            pltpu.VMEM((CHUNK,), F32),
            pltpu.VMEM((CHUNK,), I32),
            pltpu.VMEM((CHUNK,), I32),
            pltpu.VMEM((CHUNK,), F32),
            pltpu.VMEM((128,), F32),
            pltpu.VMEM((CHUNK, 128), F32),
            pltpu.VMEM((CHUNK, 128), F32),
            pltpu.VMEM((CHUNK, 128), F32),
            pltpu.VMEM((CHUNK, 128), F32),
            pltpu.VMEM((CHUNK, 128), F32),
            pltpu.VMEM((CHUNK, 128), F32),
            pltpu.SemaphoreType.DMA,
            pltpu.SemaphoreType.DMA,
            pltpu.SemaphoreType.DMA,
            pltpu.SemaphoreType.DMA,
            pltpu.SemaphoreType.DMA,
            pltpu.SemaphoreType.DMA,
            pltpu.SemaphoreType.DMA,
            pltpu.SemaphoreType.DMA,
        ],
        compiler_params=pltpu.CompilerParams(use_tc_tiling_on_sc=False),
    )
    def gather_kernel(row_h, col_h, dist_h, pr_h, pc_h, wd_h, pre_h,
                      ir_a, ic_a, id_a, ir_b, ic_b, id_b, wdv,
                      a_r, a_c, b_r, b_c, o_a, o_b,
                      s_ia, s_ib, s_ar, s_ac, s_br, s_bc, s_wa, s_wb):
        cid = lax.axis_index("c")
        sid = lax.axis_index("s")
        nchunk = jnp.where(cid == 0, h0, h1)
        ng = nchunk // 2
        base0 = jnp.where(cid == 0, sid * h0, NS * h0 + sid * h1) * CHUNK
        pltpu.sync_copy(wd_h, wdv)

        def fire_rc(t, ir, ic, si):
            sl = pl.ds(base0 + t * CHUNK, CHUNK)
            pltpu.async_copy(row_h.at[sl], ir, si)
            pltpu.async_copy(col_h.at[sl], ic, si)

        def wait_rc(t, ir, ic, si):
            sl = pl.ds(base0 + t * CHUNK, CHUNK)
            pltpu.make_async_copy(row_h.at[sl], ir, si).wait()
            pltpu.make_async_copy(col_h.at[sl], ic, si).wait()

        def fire_d(t, idv, si):
            sl = pl.ds(base0 + t * CHUNK, CHUNK)
            pltpu.async_copy(dist_h.at[sl], idv, si)

        def wait_d(t, idv, si):
            sl = pl.ds(base0 + t * CHUNK, CHUNK)
            pltpu.make_async_copy(dist_h.at[sl], idv, si).wait()

        def fire_g(ir, ic, br, bc, sr, sc):
            pltpu.async_copy(pr_h.at[ir], br, sr)
            pltpu.async_copy(pc_h.at[ic], bc, sc)

        def waitg(ir, ic, br, bc, sr, sc):
            pltpu.make_async_copy(pr_h.at[ir], br, sr).wait()
            pltpu.make_async_copy(pc_h.at[ic], bc, sc).wait()

        def fma(idv, br, bc, o):
            def grp(g, c2):
                dv = idv[pl.ds(g * 16, 16)]
                for r in range(16):
                    i = g * 16 + r
                    d = dv[r]
                    for j in range(8):
                        sl = pl.ds(j * 16, 16)
                        o[i, sl] = br[i, sl] + bc[i, sl] + d * wdv[sl]
                return c2

            lax.fori_loop(0, CHUNK // 16, grp, 0)

        def firew(t, o, sw):
            pltpu.async_copy(o, pre_h.at[pl.ds(base0 + t * CHUNK, CHUNK)], sw)

        def waitw(t, o, sw):
            pltpu.make_async_copy(
                o, pre_h.at[pl.ds(base0 + t * CHUNK, CHUNK)], sw).wait()

        seta = (ir_a, ic_a, id_a, s_ia, a_r, a_c, s_ar, s_ac)
        setb = (ir_b, ic_b, id_b, s_ib, b_r, b_c, s_br, s_bc)

        def step(t, st, o, sw, first, last):
            ir, ic, idv, si, br, bc, sr, sc = st
            waitg(ir, ic, br, bc, sr, sc)
            if not last:
                fire_rc(t + 2, ir, ic, si)
            if not first:
                waitw(t - 2, o, sw)
            wait_d(t, idv, si)
            fma(idv, br, bc, o)
            if not last:
                wait_rc(t + 2, ir, ic, si)
                fire_g(ir, ic, br, bc, sr, sc)
                fire_d(t + 2, idv, si)
            firew(t, o, sw)

        fire_rc(0, ir_a, ic_a, s_ia)
        fire_d(0, id_a, s_ia)
        fire_rc(1, ir_b, ic_b, s_ib)
        fire_d(1, id_b, s_ib)
        wait_rc(0, ir_a, ic_a, s_ia)
        fire_g(ir_a, ic_a, a_r, a_c, s_ar, s_ac)
        wait_rc(1, ir_b, ic_b, s_ib)
        fire_g(ir_b, ic_b, b_r, b_c, s_br, s_bc)
        step(0, seta, o_a, s_wa, True, False)
        step(1, setb, o_b, s_wb, True, False)

        def body(g, carry):
            step(2 * g, seta, o_a, s_wa, False, False)
            step(2 * g + 1, setb, o_b, s_wb, False, False)
            return carry

        lax.fori_loop(1, ng - 1, body, 0)

        step(nchunk - 2, seta, o_a, s_wa, False, True)
        step(nchunk - 1, setb, o_b, s_wb, False, True)
        waitw(nchunk - 2, o_a, s_wa)
        waitw(nchunk - 1, o_b, s_wb)

    return gather_kernel


def _edge_body(pre_ref, be1_ref, we2_ref, be2_ref, wa_ref, ba_ref,
               wc1_ref, bc1_ref, wc2t_ref, msg_ref, cwt_ref):
    t = _silu(pre_ref[...] + be1_ref[...])
    m = jnp.dot(t, we2_ref[...], preferred_element_type=F32) + be2_ref[...]
    a = jax.nn.sigmoid(jnp.dot(m, wa_ref[...], preferred_element_type=F32)
                       + ba_ref[...])
    msg = m * a
    msg_ref[...] = msg
    c = _silu(jnp.dot(msg, wc1_ref[...], preferred_element_type=F32)
              + bc1_ref[...])
    cwt_ref[...] = jax.lax.dot_general(
        wc2t_ref[...], c, (((1,), (1,)), ((), ())),
        preferred_element_type=F32)


def _edge_mlp(pre, be1, We2, be2, Wa, ba, Wc1, bc1, Wc2t, epad):
    be = 4096
    grid = epad // be
    full = lambda i: (0, 0)
    return pl.pallas_call(
        _edge_body,
        grid=(grid,),
        in_specs=[
            pl.BlockSpec((be, 128), lambda i: (i, 0)),
            pl.BlockSpec((1, 128), full),
            pl.BlockSpec((128, 128), full),
            pl.BlockSpec((1, 128), full),
            pl.BlockSpec((128, 1), full),
            pl.BlockSpec((1, 1), full),
            pl.BlockSpec((128, 128), full),
            pl.BlockSpec((1, 128), full),
            pl.BlockSpec((1, 128), full),
        ],
        out_specs=[
            pl.BlockSpec((be, 128), lambda i: (i, 0)),
            pl.BlockSpec((1, be), lambda i: (0, i)),
        ],
        out_shape=[
            jax.ShapeDtypeStruct((epad, 128), F32),
            jax.ShapeDtypeStruct((1, epad), F32),
        ],
    )(pre, be1, We2, be2, Wa, ba, Wc1, bc1, Wc2t)


def _make_scatter(epad, npad):
    ept = epad // NS
    nchunk = ept // CHUNK
    rpt = npad // NS

    @functools.partial(
        pl.kernel,
        out_type=(
            jax.ShapeDtypeStruct((npad, 64), F32),
            jax.ShapeDtypeStruct((npad, 64), F32),
        ),
        mesh=_mesh(),
        scratch_types=[
            pltpu.VMEM((CHUNK,), I32),
            pltpu.VMEM((CHUNK, 64), F32),
            pltpu.VMEM((CHUNK,), I32),
            pltpu.VMEM((CHUNK, 64), F32),
            pltpu.VMEM((rpt, 64), F32),
            pltpu.VMEM_SHARED((npad, 64), F32),
            pltpu.SemaphoreType.DMA,
            pltpu.SemaphoreType.DMA,
        ],
        compiler_params=pltpu.CompilerParams(use_tc_tiling_on_sc=False),
    )
    def scatter_kernel(col_h, msg_h, agga_h, aggb_h,
                       ic_a, mb_a, ic_b, mb_b, z64, agg_sh, s_a, s_b):
        cid = lax.axis_index("c")
        sid = lax.axis_index("s")
        zv = jnp.zeros((16,), F32)

        def zrow(i, carry):
            for j in range(4):
                z64[i, pl.ds(j * 16, 16)] = zv
            return carry

        lax.fori_loop(0, rpt, zrow, 0)
        pltpu.sync_copy(z64, agg_sh.at[pl.ds(sid * rpt, rpt)])
        plsc.subcore_barrier()

        base0 = sid * ept

        def fire(t, ic, mb, si):
            sl = pl.ds(base0 + t * CHUNK, CHUNK)
            pltpu.async_copy(col_h.at[sl], ic, si)
            pltpu.async_copy(msg_h.at[sl, pl.ds(cid * 64, 64)], mb, si)

        def waitld(t, ic, mb, si):
            sl = pl.ds(base0 + t * CHUNK, CHUNK)
            pltpu.make_async_copy(col_h.at[sl], ic, si).wait()
            pltpu.make_async_copy(msg_h.at[sl, pl.ds(cid * 64, 64)], mb,
                                  si).wait()

        def work(t, ic, mb, si, last):
            waitld(t, ic, mb, si)
            pltpu.sync_copy(mb, agg_sh.at[ic], add=True)
            if not last:
                fire(t + 2, ic, mb, si)

        fire(0, ic_a, mb_a, s_a)
        fire(1, ic_b, mb_b, s_b)

        def body(g, carry):
            work(2 * g, ic_a, mb_a, s_a, False)
            work(2 * g + 1, ic_b, mb_b, s_b, False)
            return carry

        lax.fori_loop(0, nchunk // 2 - 1, body, 0)
        work(nchunk - 2, ic_a, mb_a, s_a, True)
        work(nchunk - 1, ic_b, mb_b, s_b, True)
        plsc.subcore_barrier()
        rows = pl.ds(sid * rpt, rpt)

        @pl.when(cid == 0)
        def _():
            pltpu.sync_copy(agg_sh.at[rows], agga_h.at[rows])

        @pl.when(cid == 1)
        def _():
            pltpu.sync_copy(agg_sh.at[rows], aggb_h.at[rows])

    return scatter_kernel


def _make_coord(epad, npad):
    ept = epad // NS
    nchunk = ept // CHUNK
    half = nchunk // 2
    rpt = npad // NS

    @functools.partial(
        pl.kernel,
        out_type=(
            jax.ShapeDtypeStruct((npad, 16), F32),
            jax.ShapeDtypeStruct((npad, 16), F32),
        ),
        mesh=_mesh(),
        scratch_types=[
            pltpu.VMEM((CHUNK,), I32),
            pltpu.VMEM((CHUNK,), I32),
            pltpu.VMEM((CHUNK,), F32),
            pltpu.VMEM((CHUNK, 16), F32),
            pltpu.VMEM((CHUNK,), I32),
            pltpu.VMEM((CHUNK,), I32),
            pltpu.VMEM((CHUNK,), F32),
            pltpu.VMEM((CHUNK, 16), F32),
            pltpu.VMEM((rpt, 16), F32),
            pltpu.VMEM((npad,), F32),
            pltpu.VMEM((npad,), F32),
            pltpu.VMEM((npad,), F32),
            pltpu.VMEM_SHARED((npad, 16), F32),
            pltpu.SemaphoreType.DMA,
            pltpu.SemaphoreType.DMA,
        ],
        compiler_params=pltpu.CompilerParams(use_tc_tiling_on_sc=False,
                                             needs_layout_passes=False),
    )
    def coord_kernel(row_h, col_h, cw_h, x3_h, xda_h, xdb_h,
                     ic_a, ir_a, cw_a, cb_a, ic_b, ir_b, cw_b, cb_b,
                     z16, xv0, xv1, xv2, xd_sh, s_a, s_b):
        cid = lax.axis_index("c")
        sid = lax.axis_index("s")
        zv = jnp.zeros((16,), F32)

        def zrow(i, carry):
            z16[i, pl.ds(0, 16)] = zv
            return carry

        lax.fori_loop(0, rpt, zrow, 0)

        def zcb(i, carry):
            cb_a[i, pl.ds(0, 16)] = zv
            cb_b[i, pl.ds(0, 16)] = zv
            return carry

        lax.fori_loop(0, CHUNK, zcb, 0)
        pltpu.sync_copy(x3_h.at[0], xv0)
        pltpu.sync_copy(x3_h.at[1], xv1)
        pltpu.sync_copy(x3_h.at[2], xv2)
        pltpu.sync_copy(z16, xd_sh.at[pl.ds(sid * rpt, rpt)])
        plsc.subcore_barrier()

        base0 = sid * ept + cid * half * CHUNK
        xvs = (xv0, xv1, xv2)

        def fire_i(t, ic, ir, cwv, si):
            sl = pl.ds(base0 + t * CHUNK, CHUNK)
            pltpu.async_copy(col_h.at[sl], ic, si)
            pltpu.async_copy(row_h.at[sl], ir, si)
            pltpu.async_copy(cw_h.at[sl], cwv, si)

        def wait_i(t, ic, ir, cwv, si):
            sl = pl.ds(base0 + t * CHUNK, CHUNK)
            pltpu.make_async_copy(col_h.at[sl], ic, si).wait()
            pltpu.make_async_copy(row_h.at[sl], ir, si).wait()
            pltpu.make_async_copy(cw_h.at[sl], cwv, si).wait()

        def work(t, ic, ir, cwb, cbuf, si, last):
            wait_i(t, ic, ir, cwb, si)
            for g in range(8):
                sl = pl.ds(g * 16, 16)
                iv_r = ir[sl]
                iv_c = ic[sl]
                cwv = cwb[sl]
                rows = lax.iota(I32, 16) + g * 16
                for k in range(3):
                    xr = plsc.load_gather(xvs[k], [iv_r])
                    xc = plsc.load_gather(xvs[k], [iv_c])
                    cols = jnp.full((16,), k, I32)
                    plsc.store_scatter(cbuf, [rows, cols], (xr - xc) * cwv)
            pltpu.sync_copy(cbuf, xd_sh.at[ic], add=True)
            if not last:
                fire_i(t + 2, ic, ir, cwb, si)

        fire_i(0, ic_a, ir_a, cw_a, s_a)
        fire_i(1, ic_b, ir_b, cw_b, s_b)

        def body(g, carry):
            t0 = 2 * g
            work(t0, ic_a, ir_a, cw_a, cb_a, s_a, False)
            work(t0 + 1, ic_b, ir_b, cw_b, cb_b, s_b, False)
            return carry

        lax.fori_loop(0, half // 2 - 1, body, 0)
        work(half - 2, ic_a, ir_a, cw_a, cb_a, s_a, True)
        work(half - 1, ic_b, ir_b, cw_b, cb_b, s_b, True)
        plsc.subcore_barrier()
        rows = pl.ds(sid * rpt, rpt)

        @pl.when(cid == 0)
        def _():
            pltpu.sync_copy(xd_sh.at[rows], xda_h.at[rows])

        @pl.when(cid == 1)
        def _():
            pltpu.sync_copy(xd_sh.at[rows], xdb_h.at[rows])

    return coord_kernel


def _node_h_body(h_ref, agga_ref, aggb_ref, wn1h_ref, wn1a_ref, bn1_ref,
                 wn2_ref, bn2_ref, g_ref, b_ref, hnew_ref):
    h = h_ref[...]
    agg = jnp.concatenate([agga_ref[...], aggb_ref[...]], axis=-1)
    t = (jnp.dot(h, wn1h_ref[...], preferred_element_type=F32)
         + jnp.dot(agg, wn1a_ref[...], preferred_element_type=F32)
         + bn1_ref[...])
    t = _silu(t)
    mid = jnp.dot(t, wn2_ref[...], preferred_element_type=F32) + bn2_ref[...]
    y = h + mid
    mu = jnp.mean(y, axis=-1, keepdims=True)
    var = jnp.mean((y - mu) ** 2, axis=-1, keepdims=True)
    hnew_ref[...] = (y - mu) / jnp.sqrt(var + 1e-5) * g_ref[...] + b_ref[...]


def _node_h(h, agga, aggb, Wn1h, Wn1a, bn1, Wn2, bn2, g, b, n):
    bn = 2000
    grid = n // bn
    full = lambda i: (0, 0)
    return pl.pallas_call(
        _node_h_body,
        grid=(grid,),
        in_specs=[
            pl.BlockSpec((bn, 128), lambda i: (i, 0)),
            pl.BlockSpec((bn, 64), lambda i: (i, 0)),
            pl.BlockSpec((bn, 64), lambda i: (i, 0)),
            pl.BlockSpec((128, 128), full),
            pl.BlockSpec((128, 128), full),
            pl.BlockSpec((1, 128), full),
            pl.BlockSpec((128, 128), full),
            pl.BlockSpec((1, 128), full),
            pl.BlockSpec((1, 128), full),
            pl.BlockSpec((1, 128), full),
        ],
        out_specs=pl.BlockSpec((bn, 128), lambda i: (i, 0)),
        out_shape=jax.ShapeDtypeStruct((n, 128), F32),
    )(h, agga, aggb, Wn1h, Wn1a, bn1, Wn2, bn2, g, b)


def _node_x_body(x16_ref, xda_ref, xdb_ref, xnew_ref):
    xnew_ref[...] = x16_ref[...] + xda_ref[...] + xdb_ref[...]


def _node_x(x16, xda, xdb, n):
    bn = 2000
    grid = n // bn
    return pl.pallas_call(
        _node_x_body,
        grid=(grid,),
        in_specs=[
            pl.BlockSpec((bn, 16), lambda i: (i, 0)),
            pl.BlockSpec((bn, 16), lambda i: (i, 0)),
            pl.BlockSpec((bn, 16), lambda i: (i, 0)),
        ],
        out_specs=pl.BlockSpec((bn, 16), lambda i: (i, 0)),
        out_shape=jax.ShapeDtypeStruct((n, 16), F32),
    )(x16, xda, xdb)


def kernel(h, x, edge_index, edge_dist, W_e1, b_e1, W_e2, b_e2, W_a, b_a,
           W_n1, b_n1, W_n2, b_n2, W_c1, b_c1, W_c2, ln_g, ln_b):
    n, d = h.shape
    e = edge_index.shape[1]
    assert d == 128

    npad = ((n + 1 + 127) // 128) * 128
    epad = ((e + 2 * NW * CHUNK - 1) // (2 * NW * CHUNK)) * (2 * NW * CHUNK)
    epw = epad // NW

    row = edge_index[0].astype(I32)
    col = edge_index[1].astype(I32)
    pad_e = epad - e
    row_p = jnp.concatenate([row, jnp.full((pad_e,), n, I32)])
    col_p = jnp.concatenate([col, jnp.full((pad_e,), n, I32)])
    dist_p = jnp.concatenate([edge_dist, jnp.zeros((pad_e,), F32)])

    hp = jnp.zeros((npad, 128), F32).at[:n].set(h)
    x16 = jnp.zeros((n, 16), F32).at[:, :3].set(x)
    x3 = jnp.zeros((3, npad), F32).at[:, :n].set(x.T)

    W_r = W_e1[:128]
    W_c = W_e1[128:256]
    w_d = W_e1[256]

    P_r, P_c = _precompute(hp, W_r, W_c, npad)

    pre = _make_gather(epad, epw)(row_p, col_p, dist_p, P_r, P_c, w_d)

    msg, cwt = _edge_mlp(
        pre, b_e1[None, :], W_e2, b_e2[None, :], W_a, b_a[None, :],
        W_c1, b_c1[None, :], W_c2.reshape(1, 128), epad)

    agga, aggb = _make_scatter(epad, npad)(col_p, msg)
    xda, xdb = _make_coord(epad, npad)(row_p, col_p, cwt[0], x3)

    h_new = _node_h(
        h, agga[:n], aggb[:n], W_n1[:128], W_n1[128:], b_n1[None, :],
        W_n2, b_n2[None, :], ln_g[None, :], ln_b[None, :], n)
    x_new16 = _node_x(x16, xda[:n], xdb[:n], n)

    return (h_new, x_new16[:, :3])


# speedup vs baseline: 5.1873x; 1.0047x over previous
import functools

import jax
import jax.numpy as jnp
from jax import lax
from jax.experimental import pallas as pl
from jax.experimental.pallas import tpu as pltpu
from jax.experimental.pallas import tpu_sc as plsc

F32 = jnp.float32
I32 = jnp.int32

NC = 2
NS = 16
NW = NC * NS
CHUNK = 128


def _silu(v):
    return v * jax.nn.sigmoid(v)


def _mesh():
    return plsc.VectorSubcoreMesh(
        core_axis_name="c", subcore_axis_name="s", num_cores=NC, num_subcores=NS)


def _pre_body(hp_ref, wr_ref, wc_ref, pr_ref, pc_ref):
    hblk = hp_ref[...]
    pr_ref[...] = jnp.dot(hblk, wr_ref[...], preferred_element_type=F32)
    pc_ref[...] = jnp.dot(hblk, wc_ref[...], preferred_element_type=F32)


def _precompute(hp, W_r, W_c, npad):
    bh = npad // 8
    return pl.pallas_call(
        _pre_body,
        grid=(8,),
        in_specs=[
            pl.BlockSpec((bh, 128), lambda i: (i, 0)),
            pl.BlockSpec((128, 128), lambda i: (0, 0)),
            pl.BlockSpec((128, 128), lambda i: (0, 0)),
        ],
        out_specs=[
            pl.BlockSpec((bh, 128), lambda i: (i, 0)),
            pl.BlockSpec((bh, 128), lambda i: (i, 0)),
        ],
        out_shape=[
            jax.ShapeDtypeStruct((npad, 128), F32),
            jax.ShapeDtypeStruct((npad, 128), F32),
        ],
    )(hp, W_r, W_c)


def _make_gather(epad, epw):
    total = epad // CHUNK
    h0 = int(total // NS * 0.59) // 2 * 2
    h1 = total // NS - h0
    assert h0 % 2 == 0 and h1 % 2 == 0 and h0 >= 6 and h1 >= 6

    @functools.partial(
        pl.kernel,
        out_type=jax.ShapeDtypeStruct((epad, 128), F32),
        mesh=_mesh(),
        scratch_types=[
            pltpu.VMEM((CHUNK,), I32),
            pltpu.VMEM((CHUNK,), I32),
            pltpu.VMEM((CHUNK,), F32),
            pltpu.VMEM((CHUNK,), I32),
            pltpu.VMEM((CHUNK,), I32),
            pltpu.VMEM((CHUNK,), F32),
            pltpu.VMEM((128,), F32),
            pltpu.VMEM((CHUNK, 128), F32),
            pltpu.VMEM((CHUNK, 128), F32),
            pltpu.VMEM((CHUNK, 128), F32),
            pltpu.VMEM((CHUNK, 128), F32),
            pltpu.VMEM((CHUNK, 128), F32),
            pltpu.VMEM((CHUNK, 128), F32),
            pltpu.SemaphoreType.DMA,
            pltpu.SemaphoreType.DMA,
            pltpu.SemaphoreType.DMA,
            pltpu.SemaphoreType.DMA,
            pltpu.SemaphoreType.DMA,
            pltpu.SemaphoreType.DMA,
            pltpu.SemaphoreType.DMA,
            pltpu.SemaphoreType.DMA,
        ],
        compiler_params=pltpu.CompilerParams(use_tc_tiling_on_sc=False),
    )
    def gather_kernel(row_h, col_h, dist_h, pr_h, pc_h, wd_h, pre_h,
                      ir_a, ic_a, id_a, ir_b, ic_b, id_b, wdv,
                      a_r, a_c, b_r, b_c, o_a, o_b,
                      s_ia, s_ib, s_ar, s_ac, s_br, s_bc, s_wa, s_wb):
        cid = lax.axis_index("c")
        sid = lax.axis_index("s")
        nchunk = jnp.where(cid == 0, h0, h1)
        ng = nchunk // 2
        base0 = jnp.where(cid == 0, sid * h0, NS * h0 + sid * h1) * CHUNK
        pltpu.sync_copy(wd_h, wdv)

        def fire_rc(t, ir, ic, si):
            sl = pl.ds(base0 + t * CHUNK, CHUNK)
            pltpu.async_copy(row_h.at[sl], ir, si)
            pltpu.async_copy(col_h.at[sl], ic, si)

        def wait_rc(t, ir, ic, si):
            sl = pl.ds(base0 + t * CHUNK, CHUNK)
            pltpu.make_async_copy(row_h.at[sl], ir, si).wait()
            pltpu.make_async_copy(col_h.at[sl], ic, si).wait()

        def fire_d(t, idv, si):
            sl = pl.ds(base0 + t * CHUNK, CHUNK)
            pltpu.async_copy(dist_h.at[sl], idv, si)

        def wait_d(t, idv, si):
            sl = pl.ds(base0 + t * CHUNK, CHUNK)
            pltpu.make_async_copy(dist_h.at[sl], idv, si).wait()

        def fire_g(ir, ic, br, bc, sr, sc):
            pltpu.async_copy(pr_h.at[ir], br, sr)
            pltpu.async_copy(pc_h.at[ic], bc, sc)

        def waitg(ir, ic, br, bc, sr, sc):
            pltpu.make_async_copy(pr_h.at[ir], br, sr).wait()
            pltpu.make_async_copy(pc_h.at[ic], bc, sc).wait()

        def fma(idv, br, bc, o):
            def grp(g, c2):
                dv = idv[pl.ds(g * 16, 16)]
                for r in range(16):
                    i = g * 16 + r
                    d = dv[r]
                    for j in range(8):
                        sl = pl.ds(j * 16, 16)
                        o[i, sl] = br[i, sl] + bc[i, sl] + d * wdv[sl]
                return c2

            lax.fori_loop(0, CHUNK // 16, grp, 0)

        def firew(t, o, sw):
            pltpu.async_copy(o, pre_h.at[pl.ds(base0 + t * CHUNK, CHUNK)], sw)

        def waitw(t, o, sw):
            pltpu.make_async_copy(
                o, pre_h.at[pl.ds(base0 + t * CHUNK, CHUNK)], sw).wait()

        seta = (ir_a, ic_a, id_a, s_ia, a_r, a_c, s_ar, s_ac)
        setb = (ir_b, ic_b, id_b, s_ib, b_r, b_c, s_br, s_bc)

        def step(t, st, o, sw, first, last):
            ir, ic, idv, si, br, bc, sr, sc = st
            waitg(ir, ic, br, bc, sr, sc)
            if not last:
                fire_rc(t + 2, ir, ic, si)
            if not first:
                waitw(t - 2, o, sw)
            wait_d(t, idv, si)
            fma(idv, br, bc, o)
            if not last:
                wait_rc(t + 2, ir, ic, si)
                fire_g(ir, ic, br, bc, sr, sc)
                fire_d(t + 2, idv, si)
            firew(t, o, sw)

        fire_rc(0, ir_a, ic_a, s_ia)
        fire_d(0, id_a, s_ia)
        fire_rc(1, ir_b, ic_b, s_ib)
        fire_d(1, id_b, s_ib)
        wait_rc(0, ir_a, ic_a, s_ia)
        fire_g(ir_a, ic_a, a_r, a_c, s_ar, s_ac)
        wait_rc(1, ir_b, ic_b, s_ib)
        fire_g(ir_b, ic_b, b_r, b_c, s_br, s_bc)
        step(0, seta, o_a, s_wa, True, False)
        step(1, setb, o_b, s_wb, True, False)

        def body(g, carry):
            step(2 * g, seta, o_a, s_wa, False, False)
            step(2 * g + 1, setb, o_b, s_wb, False, False)
            return carry

        lax.fori_loop(1, ng - 1, body, 0)

        step(nchunk - 2, seta, o_a, s_wa, False, True)
        step(nchunk - 1, setb, o_b, s_wb, False, True)
        waitw(nchunk - 2, o_a, s_wa)
        waitw(nchunk - 1, o_b, s_wb)

    return gather_kernel


def _edge_body(pre_ref, be1_ref, we2_ref, be2_ref, wa_ref, ba_ref,
               wc1_ref, bc1_ref, wc2t_ref, msg_ref, cwt_ref):
    t = _silu(pre_ref[...] + be1_ref[...])
    m = jnp.dot(t, we2_ref[...], preferred_element_type=F32) + be2_ref[...]
    a = jax.nn.sigmoid(jnp.dot(m, wa_ref[...], preferred_element_type=F32)
                       + ba_ref[...])
    msg = m * a
    msg_ref[...] = msg
    c = _silu(jnp.dot(msg, wc1_ref[...], preferred_element_type=F32)
              + bc1_ref[...])
    cwt_ref[...] = jax.lax.dot_general(
        wc2t_ref[...], c, (((1,), (1,)), ((), ())),
        preferred_element_type=F32)


def _edge_mlp(pre, be1, We2, be2, Wa, ba, Wc1, bc1, Wc2t, epad):
    be = 4096
    grid = epad // be
    full = lambda i: (0, 0)
    return pl.pallas_call(
        _edge_body,
        grid=(grid,),
        in_specs=[
            pl.BlockSpec((be, 128), lambda i: (i, 0)),
            pl.BlockSpec((1, 128), full),
            pl.BlockSpec((128, 128), full),
            pl.BlockSpec((1, 128), full),
            pl.BlockSpec((128, 1), full),
            pl.BlockSpec((1, 1), full),
            pl.BlockSpec((128, 128), full),
            pl.BlockSpec((1, 128), full),
            pl.BlockSpec((1, 128), full),
        ],
        out_specs=[
            pl.BlockSpec((be, 128), lambda i: (i, 0)),
            pl.BlockSpec((1, be), lambda i: (0, i)),
        ],
        out_shape=[
            jax.ShapeDtypeStruct((epad, 128), F32),
            jax.ShapeDtypeStruct((1, epad), F32),
        ],
    )(pre, be1, We2, be2, Wa, ba, Wc1, bc1, Wc2t)


def _make_scatter(epad, npad):
    ept = epad // NS
    nchunk = ept // CHUNK
    rpt = npad // NS

    @functools.partial(
        pl.kernel,
        out_type=(
            jax.ShapeDtypeStruct((npad, 64), F32),
            jax.ShapeDtypeStruct((npad, 64), F32),
        ),
        mesh=_mesh(),
        scratch_types=[
            pltpu.VMEM((CHUNK,), I32),
            pltpu.VMEM((CHUNK, 64), F32),
            pltpu.VMEM((CHUNK,), I32),
            pltpu.VMEM((CHUNK, 64), F32),
            pltpu.VMEM((rpt, 64), F32),
            pltpu.VMEM_SHARED((npad, 64), F32),
            pltpu.SemaphoreType.DMA,
            pltpu.SemaphoreType.DMA,
        ],
        compiler_params=pltpu.CompilerParams(use_tc_tiling_on_sc=False),
    )
    def scatter_kernel(col_h, msg_h, agga_h, aggb_h,
                       ic_a, mb_a, ic_b, mb_b, z64, agg_sh, s_a, s_b):
        cid = lax.axis_index("c")
        sid = lax.axis_index("s")
        zv = jnp.zeros((16,), F32)

        def zrow(i, carry):
            for j in range(4):
                z64[i, pl.ds(j * 16, 16)] = zv
            return carry

        lax.fori_loop(0, rpt, zrow, 0)
        pltpu.sync_copy(z64, agg_sh.at[pl.ds(sid * rpt, rpt)])
        plsc.subcore_barrier()

        base0 = sid * ept

        def fire(t, ic, mb, si):
            sl = pl.ds(base0 + t * CHUNK, CHUNK)
            pltpu.async_copy(col_h.at[sl], ic, si)
            pltpu.async_copy(msg_h.at[sl, pl.ds(cid * 64, 64)], mb, si)

        def waitld(t, ic, mb, si):
            sl = pl.ds(base0 + t * CHUNK, CHUNK)
            pltpu.make_async_copy(col_h.at[sl], ic, si).wait()
            pltpu.make_async_copy(msg_h.at[sl, pl.ds(cid * 64, 64)], mb,
                                  si).wait()

        def work(t, ic, mb, si, last):
            waitld(t, ic, mb, si)
            pltpu.sync_copy(mb, agg_sh.at[ic], add=True)
            if not last:
                fire(t + 2, ic, mb, si)

        fire(0, ic_a, mb_a, s_a)
        fire(1, ic_b, mb_b, s_b)

        def body(g, carry):
            work(2 * g, ic_a, mb_a, s_a, False)
            work(2 * g + 1, ic_b, mb_b, s_b, False)
            return carry

        lax.fori_loop(0, nchunk // 2 - 1, body, 0)
        work(nchunk - 2, ic_a, mb_a, s_a, True)
        work(nchunk - 1, ic_b, mb_b, s_b, True)
        plsc.subcore_barrier()
        rows = pl.ds(sid * rpt, rpt)

        @pl.when(cid == 0)
        def _():
            pltpu.sync_copy(agg_sh.at[rows], agga_h.at[rows])

        @pl.when(cid == 1)
        def _():
            pltpu.sync_copy(agg_sh.at[rows], aggb_h.at[rows])

    return scatter_kernel


def _make_coord(epad, npad):
    ept = epad // NS
    nchunk = ept // CHUNK
    half = nchunk // 2
    rpt = npad // NS

    @functools.partial(
        pl.kernel,
        out_type=(
            jax.ShapeDtypeStruct((npad, 16), F32),
            jax.ShapeDtypeStruct((npad, 16), F32),
        ),
        mesh=_mesh(),
        scratch_types=[
            pltpu.VMEM((CHUNK,), I32),
            pltpu.VMEM((CHUNK,), I32),
            pltpu.VMEM((CHUNK,), F32),
            pltpu.VMEM((CHUNK, 16), F32),
            pltpu.VMEM((CHUNK,), I32),
            pltpu.VMEM((CHUNK,), I32),
            pltpu.VMEM((CHUNK,), F32),
            pltpu.VMEM((CHUNK, 16), F32),
            pltpu.VMEM((rpt, 16), F32),
            pltpu.VMEM((npad,), F32),
            pltpu.VMEM((npad,), F32),
            pltpu.VMEM((npad,), F32),
            pltpu.VMEM_SHARED((npad, 16), F32),
            pltpu.SemaphoreType.DMA,
            pltpu.SemaphoreType.DMA,
        ],
        compiler_params=pltpu.CompilerParams(use_tc_tiling_on_sc=False,
                                             needs_layout_passes=False),
    )
    def coord_kernel(row_h, col_h, cw_h, x3_h, xda_h, xdb_h,
                     ic_a, ir_a, cw_a, cb_a, ic_b, ir_b, cw_b, cb_b,
                     z16, xv0, xv1, xv2, xd_sh, s_a, s_b):
        cid = lax.axis_index("c")
        sid = lax.axis_index("s")
        zv = jnp.zeros((16,), F32)

        def zrow(i, carry):
            z16[i, pl.ds(0, 16)] = zv
            return carry

        lax.fori_loop(0, rpt, zrow, 0)

        def zcb(i, carry):
            cb_a[i, pl.ds(0, 16)] = zv
            cb_b[i, pl.ds(0, 16)] = zv
            return carry

        lax.fori_loop(0, CHUNK, zcb, 0)
        pltpu.sync_copy(x3_h.at[0], xv0)
        pltpu.sync_copy(x3_h.at[1], xv1)
        pltpu.sync_copy(x3_h.at[2], xv2)
        pltpu.sync_copy(z16, xd_sh.at[pl.ds(sid * rpt, rpt)])
        plsc.subcore_barrier()

        base0 = sid * ept + cid * half * CHUNK
        xvs = (xv0, xv1, xv2)

        def fire_i(t, ic, ir, cwv, si):
            sl = pl.ds(base0 + t * CHUNK, CHUNK)
            pltpu.async_copy(col_h.at[sl], ic, si)
            pltpu.async_copy(row_h.at[sl], ir, si)
            pltpu.async_copy(cw_h.at[sl], cwv, si)

        def wait_i(t, ic, ir, cwv, si):
            sl = pl.ds(base0 + t * CHUNK, CHUNK)
            pltpu.make_async_copy(col_h.at[sl], ic, si).wait()
            pltpu.make_async_copy(row_h.at[sl], ir, si).wait()
            pltpu.make_async_copy(cw_h.at[sl], cwv, si).wait()

        def work(t, ic, ir, cwb, cbuf, si, last):
            wait_i(t, ic, ir, cwb, si)
            for g in range(8):
                sl = pl.ds(g * 16, 16)
                iv_r = ir[sl]
                iv_c = ic[sl]
                cwv = cwb[sl]
                rows = lax.iota(I32, 16) + g * 16
                for k in range(3):
                    xr = plsc.load_gather(xvs[k], [iv_r])
                    xc = plsc.load_gather(xvs[k], [iv_c])
                    cols = jnp.full((16,), k, I32)
                    plsc.store_scatter(cbuf, [rows, cols], (xr - xc) * cwv)
            pltpu.sync_copy(cbuf, xd_sh.at[ic], add=True)
            if not last:
                fire_i(t + 2, ic, ir, cwb, si)

        fire_i(0, ic_a, ir_a, cw_a, s_a)
        fire_i(1, ic_b, ir_b, cw_b, s_b)

        def body(g, carry):
            t0 = 2 * g
            work(t0, ic_a, ir_a, cw_a, cb_a, s_a, False)
            work(t0 + 1, ic_b, ir_b, cw_b, cb_b, s_b, False)
            return carry

        lax.fori_loop(0, half // 2 - 1, body, 0)
        work(half - 2, ic_a, ir_a, cw_a, cb_a, s_a, True)
        work(half - 1, ic_b, ir_b, cw_b, cb_b, s_b, True)
        plsc.subcore_barrier()
        rows = pl.ds(sid * rpt, rpt)

        @pl.when(cid == 0)
        def _():
            pltpu.sync_copy(xd_sh.at[rows], xda_h.at[rows])

        @pl.when(cid == 1)
        def _():
            pltpu.sync_copy(xd_sh.at[rows], xdb_h.at[rows])

    return coord_kernel


def _node_h_body(h_ref, agga_ref, aggb_ref, wn1h_ref, wn1a_ref, bn1_ref,
                 wn2_ref, bn2_ref, g_ref, b_ref, hnew_ref):
    h = h_ref[...]
    agg = jnp.concatenate([agga_ref[...], aggb_ref[...]], axis=-1)
    t = (jnp.dot(h, wn1h_ref[...], preferred_element_type=F32)
         + jnp.dot(agg, wn1a_ref[...], preferred_element_type=F32)
         + bn1_ref[...])
    t = _silu(t)
    mid = jnp.dot(t, wn2_ref[...], preferred_element_type=F32) + bn2_ref[...]
    y = h + mid
    mu = jnp.mean(y, axis=-1, keepdims=True)
    var = jnp.mean((y - mu) ** 2, axis=-1, keepdims=True)
    hnew_ref[...] = (y - mu) / jnp.sqrt(var + 1e-5) * g_ref[...] + b_ref[...]


def _node_h(h, agga, aggb, Wn1h, Wn1a, bn1, Wn2, bn2, g, b, n):
    bn = 2000
    grid = n // bn
    full = lambda i: (0, 0)
    return pl.pallas_call(
        _node_h_body,
        grid=(grid,),
        in_specs=[
            pl.BlockSpec((bn, 128), lambda i: (i, 0)),
            pl.BlockSpec((bn, 64), lambda i: (i, 0)),
            pl.BlockSpec((bn, 64), lambda i: (i, 0)),
            pl.BlockSpec((128, 128), full),
            pl.BlockSpec((128, 128), full),
            pl.BlockSpec((1, 128), full),
            pl.BlockSpec((128, 128), full),
            pl.BlockSpec((1, 128), full),
            pl.BlockSpec((1, 128), full),
            pl.BlockSpec((1, 128), full),
        ],
        out_specs=pl.BlockSpec((bn, 128), lambda i: (i, 0)),
        out_shape=jax.ShapeDtypeStruct((n, 128), F32),
    )(h, agga, aggb, Wn1h, Wn1a, bn1, Wn2, bn2, g, b)


def _node_x_body(x16_ref, xda_ref, xdb_ref, xnew_ref):
    xnew_ref[...] = x16_ref[...] + xda_ref[...] + xdb_ref[...]


def _node_x(x16, xda, xdb, n):
    bn = 2000
    grid = n // bn
    return pl.pallas_call(
        _node_x_body,
        grid=(grid,),
        in_specs=[
            pl.BlockSpec((bn, 16), lambda i: (i, 0)),
            pl.BlockSpec((bn, 16), lambda i: (i, 0)),
            pl.BlockSpec((bn, 16), lambda i: (i, 0)),
        ],
        out_specs=pl.BlockSpec((bn, 16), lambda i: (i, 0)),
        out_shape=jax.ShapeDtypeStruct((n, 16), F32),
    )(x16, xda, xdb)


def kernel(h, x, edge_index, edge_dist, W_e1, b_e1, W_e2, b_e2, W_a, b_a,
           W_n1, b_n1, W_n2, b_n2, W_c1, b_c1, W_c2, ln_g, ln_b):
    n, d = h.shape
    e = edge_index.shape[1]
    assert d == 128

    npad = ((n + 1 + 127) // 128) * 128
    epad = ((e + 2 * NW * CHUNK - 1) // (2 * NW * CHUNK)) * (2 * NW * CHUNK)
    epw = epad // NW

    row = edge_index[0].astype(I32)
    col = edge_index[1].astype(I32)
    pad_e = epad - e
    row_p = jnp.concatenate([row, jnp.full((pad_e,), n, I32)])
    col_p = jnp.concatenate([col, jnp.full((pad_e,), n, I32)])
    dist_p = jnp.concatenate([edge_dist, jnp.zeros((pad_e,), F32)])

    hp = jnp.zeros((npad, 128), F32).at[:n].set(h)
    x16 = jnp.zeros((n, 16), F32).at[:, :3].set(x)
    x3 = jnp.zeros((3, npad), F32).at[:, :n].set(x.T)

    W_r = W_e1[:128]
    W_c = W_e1[128:256]
    w_d = W_e1[256]

    P_r, P_c = _precompute(hp, W_r, W_c, npad)

    pre = _make_gather(epad, epw)(row_p, col_p, dist_p, P_r, P_c, w_d)

    msg, cwt = _edge_mlp(
        pre, b_e1[None, :], W_e2, b_e2[None, :], W_a, b_a[None, :],
        W_c1, b_c1[None, :], W_c2.reshape(1, 128), epad)

    agga, aggb = _make_scatter(epad, npad)(col_p, msg)
    xda, xdb = _make_coord(epad, npad)(row_p, col_p, cwt[0], x3)

    h_new = _node_h(
        h, agga[:n], aggb[:n], W_n1[:128], W_n1[128:], b_n1[None, :],
        W_n2, b_n2[None, :], ln_g[None, :], ln_b[None, :], n)
    x_new16 = _node_x(x16, xda[:n], xdb[:n], n)

    return (h_new, x_new16[:, :3])
